# baseline (speedup 1.0000x reference)
import jax
import jax.numpy as jnp
from jax import lax
from jax.experimental import pallas as pl
from jax.experimental.pallas import tpu as pltpu
from jax.experimental.pallas import tpu_sc as plsc

NN = 10000
NP2 = 10240
DD = 128
CC = 64
NEG_SLOPE = 0.2

ER = 330000
NC, NS, LANES = 2, 16, 16
NW = NC * NS
KE = 128
NCH = 82
BW = NCH * KE
EPAD = BW * NW
NPAIR = NCH // 2
RPS = NP2 // NS

_sc_mesh = plsc.VectorSubcoreMesh(core_axis_name="c", subcore_axis_name="s")
_sc_params = pltpu.CompilerParams(needs_layout_passes=False)


def _mm_pre_kernel(x_ref, w_ref, dis_ref, o_ref):
    xw = jnp.dot(x_ref[...], w_ref[...], preferred_element_type=jnp.float32)
    o_ref[...] = xw * dis_ref[...][:, None]


def tc_matmul_pre(x, w, dis):
    return pl.pallas_call(
        _mm_pre_kernel,
        out_shape=jax.ShapeDtypeStruct((x.shape[0], w.shape[1]), jnp.float32),
    )(x, w, dis)


def _dis_kernel(d0_ref, d1_ref, o_ref):
    d = d0_ref[...][:NN] + d1_ref[...][:NN]
    o_ref[...] = jnp.where(d > 0, lax.rsqrt(d), 0.0)


def tc_dis(deg0, deg1):
    return pl.pallas_call(
        _dis_kernel,
        out_shape=jax.ShapeDtypeStruct((NN,), jnp.float32),
    )(deg0, deg1)


def _post_gcn_prep_kernel(a0_ref, a1_ref, dis_ref, b_ref, w_ref, o_ref):
    acc = (a0_ref[...] + a1_ref[...]) * dis_ref[...][:, None]
    h = jnp.maximum(acc + b_ref[...][None, :], 0.0)
    o_ref[...] = jnp.dot(h, w_ref[...], preferred_element_type=jnp.float32)


def tc_post_gcn_prep(acc0, acc1, dis, b, w):
    return pl.pallas_call(
        _post_gcn_prep_kernel,
        out_shape=jax.ShapeDtypeStruct((NN, w.shape[1]), jnp.float32),
    )(acc0, acc1, dis, b, w)


def _att_kernel(xw_ref, asrc_ref, adst_ref, as_ref, ad_ref, u_ref):
    xw = xw_ref[...]
    a_s = jnp.sum(xw * asrc_ref[...][None, :], axis=1)
    a_d = jnp.sum(xw * adst_ref[...][None, :], axis=1)
    as_ref[...] = a_s
    ad_ref[...] = a_d
    e_ub = jnp.max(a_s) + a_d
    u_ref[...] = jnp.where(e_ub >= 0, e_ub, NEG_SLOPE * e_ub)


def tc_att(xw, a_src, a_dst):
    return pl.pallas_call(
        _att_kernel,
        out_shape=[jax.ShapeDtypeStruct((NN,), jnp.float32)] * 3,
    )(xw, a_src, a_dst)


def _post_gat_prep_kernel(a0_ref, a1_ref, s0_ref, s1_ref, b_ref, w_ref,
                          dis_ref, o_ref):
    s = (s0_ref[...][:NN] + s1_ref[...][:NN])[:, None] + 1e-16
    h = jnp.maximum((a0_ref[...] + a1_ref[...]) / s + b_ref[...][None, :], 0.0)
    xw = jnp.dot(h, w_ref[...], preferred_element_type=jnp.float32)
    o_ref[...] = xw * dis_ref[...][:, None]


def tc_post_gat_prep(acc0, acc1, s0, s1, b, w, dis):
    return pl.pallas_call(
        _post_gat_prep_kernel,
        out_shape=jax.ShapeDtypeStruct((NN, w.shape[1]), jnp.float32),
    )(acc0, acc1, s0, s1, b, w, dis)


def _final_kernel(a0_ref, a1_ref, s0_ref, s1_ref, b_ref, w_ref, ob_ref,
                  h_ref, z_ref):
    s = (s0_ref[...][:NN] + s1_ref[...][:NN])[:, None] + 1e-16
    h = jnp.maximum((a0_ref[...] + a1_ref[...]) / s + b_ref[...][None, :], 0.0)
    h_ref[...] = h
    z_ref[...] = (jnp.dot(h, w_ref[...], preferred_element_type=jnp.float32)
                  + ob_ref[...][None, :])


def tc_final(acc0, acc1, s0, s1, b, w, ob):
    return pl.pallas_call(
        _final_kernel,
        out_shape=[jax.ShapeDtypeStruct((NN, DD), jnp.float32),
                   jax.ShapeDtypeStruct((NN, CC), jnp.float32)],
    )(acc0, acc1, s0, s1, b, w, ob)


def _edge_mask(eb, j):
    ids = eb + j * LANES + lax.iota(jnp.int32, LANES)
    return ids < ER


def _deg_body(dst_hbm, z1_hbm, out0, out1, didx, ones_v, deg_sh, sem):
    c = lax.axis_index("c")
    s = lax.axis_index("s")
    wid = c * NS + s
    pltpu.sync_copy(z1_hbm, deg_sh.at[pl.ds(s * RPS, RPS)])
    plsc.subcore_barrier()
    base = wid * BW

    def chunk(i, carry):
        eb = base + i * KE
        pltpu.sync_copy(dst_hbm.at[pl.ds(eb, KE)], didx)

        def grp(j, carry2):
            m = _edge_mask(eb, j)
            ones_v[pl.ds(j * LANES, LANES)] = jnp.where(m, 1.0, 0.0)
            return carry2

        lax.fori_loop(0, KE // LANES, grp, 0)
        pltpu.sync_copy(ones_v, deg_sh.at[didx], add=True)
        return carry

    lax.fori_loop(0, NCH, chunk, 0)
    plsc.subcore_barrier()
    sl = pl.ds(s * RPS, RPS)

    @pl.when(c == 0)
    def _():
        pltpu.sync_copy(deg_sh.at[sl], out0.at[sl])

    @pl.when(c == 1)
    def _():
        pltpu.sync_copy(deg_sh.at[sl], out1.at[sl])


def sc_degree(dst_p, z1):
    f = pl.kernel(
        _deg_body,
        out_type=[jax.ShapeDtypeStruct((NP2,), jnp.float32)] * 2,
        mesh=_sc_mesh,
        compiler_params=_sc_params,
        scratch_types=[
            pltpu.VMEM((KE,), jnp.int32),
            pltpu.VMEM((KE,), jnp.float32),
            pltpu.VMEM_SHARED((NP2,), jnp.float32),
            pltpu.SemaphoreType.DMA,
        ],
    )
    return f(dst_p, z1)


def _gcn_body(src_hbm, dst_hbm, xw_hbm, z2_hbm, out0, out1,
              sidx0, sidx1, didx0, didx1, rows0, rows1, acc_sh, sem0, sem1):
    c = lax.axis_index("c")
    s = lax.axis_index("s")
    wid = c * NS + s
    pltpu.sync_copy(z2_hbm, acc_sh.at[pl.ds(s * RPS, RPS)])
    plsc.subcore_barrier()
    base = wid * BW

    def pair(p, carry):
        e0 = base + 2 * p * KE
        e1 = e0 + KE
        pltpu.sync_copy(src_hbm.at[pl.ds(e0, KE)], sidx0)
        g0 = pltpu.async_copy(xw_hbm.at[sidx0], rows0, sem0)
        pltpu.sync_copy(src_hbm.at[pl.ds(e1, KE)], sidx1)
        g1 = pltpu.async_copy(xw_hbm.at[sidx1], rows1, sem1)
        pltpu.sync_copy(dst_hbm.at[pl.ds(e0, KE)], didx0)
        pltpu.sync_copy(dst_hbm.at[pl.ds(e1, KE)], didx1)
        g0.wait()
        pltpu.sync_copy(rows0, acc_sh.at[didx0], add=True)
        g1.wait()
        pltpu.sync_copy(rows1, acc_sh.at[didx1], add=True)
        return carry

    lax.fori_loop(0, NPAIR, pair, 0)
    plsc.subcore_barrier()
    sl = pl.ds(s * RPS, RPS)

    @pl.when(c == 0)
    def _():
        pltpu.sync_copy(acc_sh.at[sl], out0.at[sl])

    @pl.when(c == 1)
    def _():
        pltpu.sync_copy(acc_sh.at[sl], out1.at[sl])


def sc_gcn_pass(src_p, dst_p, xw, z2):
    f = pl.kernel(
        _gcn_body,
        out_type=[jax.ShapeDtypeStruct((NP2, DD), jnp.float32)] * 2,
        mesh=_sc_mesh,
        compiler_params=_sc_params,
        scratch_types=[
            pltpu.VMEM((KE,), jnp.int32),
            pltpu.VMEM((KE,), jnp.int32),
            pltpu.VMEM((KE,), jnp.int32),
            pltpu.VMEM((KE,), jnp.int32),
            pltpu.VMEM((KE, DD), jnp.float32),
            pltpu.VMEM((KE, DD), jnp.float32),
            pltpu.VMEM_SHARED((NP2, DD), jnp.float32),
            pltpu.SemaphoreType.DMA,
            pltpu.SemaphoreType.DMA,
        ],
    )
    return f(src_p, dst_p, xw, z2)


def _gat_w_body(src_hbm, dst_hbm, as_hbm, ad_hbm, u_hbm, z1_hbm,
                ex_out, so0, so1,
                as_v, ad_v, u_v, sidx, didx, wv, s_sh, sem):
    c = lax.axis_index("c")
    s = lax.axis_index("s")
    wid = c * NS + s
    pltpu.sync_copy(z1_hbm, s_sh.at[pl.ds(s * RPS, RPS)])
    pltpu.sync_copy(as_hbm, as_v)
    pltpu.sync_copy(ad_hbm, ad_v)
    pltpu.sync_copy(u_hbm, u_v)
    plsc.subcore_barrier()
    base = wid * BW

    def chunk(i, carry):
        eb = base + i * KE
        pltpu.sync_copy(src_hbm.at[pl.ds(eb, KE)], sidx)
        pltpu.sync_copy(dst_hbm.at[pl.ds(eb, KE)], didx)

        def grp(j, carry2):
            sv = sidx[pl.ds(j * LANES, LANES)]
            dv = didx[pl.ds(j * LANES, LANES)]
            e = plsc.load_gather(as_v, [sv]) + plsc.load_gather(ad_v, [dv])
            e = jnp.where(e >= 0, e, NEG_SLOPE * e)
            ex = jnp.exp(e - plsc.load_gather(u_v, [dv]))
            ex = jnp.where(_edge_mask(eb, j), ex, 0.0)
            wv[pl.ds(j * LANES, LANES)] = ex
            return carry2

        lax.fori_loop(0, KE // LANES, grp, 0)
        pltpu.sync_copy(wv, ex_out.at[pl.ds(eb, KE)])
        pltpu.sync_copy(wv, s_sh.at[didx], add=True)
        return carry

    lax.fori_loop(0, NCH, chunk, 0)
    plsc.subcore_barrier()
    sl = pl.ds(s * RPS, RPS)

    @pl.when(c == 0)
    def _():
        pltpu.sync_copy(s_sh.at[sl], so0.at[sl])

    @pl.when(c == 1)
    def _():
        pltpu.sync_copy(s_sh.at[sl], so1.at[sl])


def sc_gat_w(src_p, dst_p, a_s, a_d, u, z1):
    f = pl.kernel(
        _gat_w_body,
        out_type=[jax.ShapeDtypeStruct((EPAD,), jnp.float32)]
        + [jax.ShapeDtypeStruct((NP2,), jnp.float32)] * 2,
        mesh=_sc_mesh,
        compiler_params=_sc_params,
        scratch_types=[
            pltpu.VMEM((NN,), jnp.float32),
            pltpu.VMEM((NN,), jnp.float32),
            pltpu.VMEM((NN,), jnp.float32),
            pltpu.VMEM((KE,), jnp.int32),
            pltpu.VMEM((KE,), jnp.int32),
            pltpu.VMEM((KE,), jnp.float32),
            pltpu.VMEM_SHARED((NP2,), jnp.float32),
            pltpu.SemaphoreType.DMA,
        ],
    )
    return f(src_p, dst_p, a_s, a_d, u, z1)


def _gat_rows_body(src_hbm, dst_hbm, xw_hbm, wgt_hbm, z2_hbm, out0, out1,
                   sidx0, sidx1, didx0, didx1, rows0, rows1, wv0, wv1,
                   acc_sh, sem0, sem1):
    c = lax.axis_index("c")
    s = lax.axis_index("s")
    wid = c * NS + s
    pltpu.sync_copy(z2_hbm, acc_sh.at[pl.ds(s * RPS, RPS)])
    plsc.subcore_barrier()
    base = wid * BW

    def do_chunk(rows, wv, didx):
        def scale(e, carry2):
            w = plsc.load_gather(wv, [jnp.full((LANES,), e, jnp.int32)])
            for jj in range(DD // LANES):
                sl = pl.ds(jj * LANES, LANES)
                rows[e, sl] = rows[e, sl] * w
            return carry2

        lax.fori_loop(0, KE, scale, 0)
        pltpu.sync_copy(rows, acc_sh.at[didx], add=True)

    def pair(p, carry):
        e0 = base + 2 * p * KE
        e1 = e0 + KE
        pltpu.sync_copy(src_hbm.at[pl.ds(e0, KE)], sidx0)
        g0 = pltpu.async_copy(xw_hbm.at[sidx0], rows0, sem0)
        pltpu.sync_copy(src_hbm.at[pl.ds(e1, KE)], sidx1)
        g1 = pltpu.async_copy(xw_hbm.at[sidx1], rows1, sem1)
        pltpu.sync_copy(dst_hbm.at[pl.ds(e0, KE)], didx0)
        pltpu.sync_copy(dst_hbm.at[pl.ds(e1, KE)], didx1)
        pltpu.sync_copy(wgt_hbm.at[pl.ds(e0, KE)], wv0)
        pltpu.sync_copy(wgt_hbm.at[pl.ds(e1, KE)], wv1)
        g0.wait()
        do_chunk(rows0, wv0, didx0)
        g1.wait()
        do_chunk(rows1, wv1, didx1)
        return carry

    lax.fori_loop(0, NPAIR, pair, 0)
    plsc.subcore_barrier()
    sl = pl.ds(s * RPS, RPS)

    @pl.when(c == 0)
    def _():
        pltpu.sync_copy(acc_sh.at[sl], out0.at[sl])

    @pl.when(c == 1)
    def _():
        pltpu.sync_copy(acc_sh.at[sl], out1.at[sl])


def sc_gat_rows(src_p, dst_p, xw, wgt, z2):
    f = pl.kernel(
        _gat_rows_body,
        out_type=[jax.ShapeDtypeStruct((NP2, DD), jnp.float32)] * 2,
        mesh=_sc_mesh,
        compiler_params=_sc_params,
        scratch_types=[
            pltpu.VMEM((KE,), jnp.int32),
            pltpu.VMEM((KE,), jnp.int32),
            pltpu.VMEM((KE,), jnp.int32),
            pltpu.VMEM((KE,), jnp.int32),
            pltpu.VMEM((KE, DD), jnp.float32),
            pltpu.VMEM((KE, DD), jnp.float32),
            pltpu.VMEM((KE,), jnp.float32),
            pltpu.VMEM((KE,), jnp.float32),
            pltpu.VMEM_SHARED((NP2, DD), jnp.float32),
            pltpu.SemaphoreType.DMA,
            pltpu.SemaphoreType.DMA,
        ],
    )
    return f(src_p, dst_p, xw, wgt, z2)


def kernel(x, edge_index, gcn1_W, gcn1_b, gat1_W, gat1_att_src, gat1_att_dst,
           gat1_b, gcn2_W, gcn2_b, gat2_W, gat2_att_src, gat2_att_dst,
           gat2_b, out_W, out_b):
    loops = jnp.arange(NN, dtype=jnp.int32)
    spad = jnp.zeros((EPAD - ER,), jnp.int32)
    dpad = NN + (jnp.arange(EPAD - ER, dtype=jnp.int32) % (NP2 - NN))
    src_p = jnp.concatenate([edge_index[0].astype(jnp.int32), loops, spad])
    dst_p = jnp.concatenate([edge_index[1].astype(jnp.int32), loops, dpad])
    z1 = jnp.zeros((RPS,), jnp.float32)
    z2 = jnp.zeros((RPS, DD), jnp.float32)

    deg0, deg1 = sc_degree(dst_p, z1)
    dis = tc_dis(deg0, deg1)

    xw1 = tc_matmul_pre(x, gcn1_W, dis)
    a1_0, a1_1 = sc_gcn_pass(src_p, dst_p, xw1, z2)

    xw2 = tc_post_gcn_prep(a1_0[:NN], a1_1[:NN], dis, gcn1_b, gat1_W)
    as2, ad2, u2 = tc_att(xw2, gat1_att_src, gat1_att_dst)
    ex2, s2_0, s2_1 = sc_gat_w(src_p, dst_p, as2, ad2, u2, z1)
    a2_0, a2_1 = sc_gat_rows(src_p, dst_p, xw2, ex2, z2)

    xw3 = tc_post_gat_prep(a2_0[:NN], a2_1[:NN], s2_0, s2_1, gat1_b, gcn2_W,
                           dis)
    a3_0, a3_1 = sc_gcn_pass(src_p, dst_p, xw3, z2)

    xw4 = tc_post_gcn_prep(a3_0[:NN], a3_1[:NN], dis, gcn2_b, gat2_W)
    as4, ad4, u4 = tc_att(xw4, gat2_att_src, gat2_att_dst)
    ex4, s4_0, s4_1 = sc_gat_w(src_p, dst_p, as4, ad4, u4, z1)
    a4_0, a4_1 = sc_gat_rows(src_p, dst_p, xw4, ex4, z2)

    h, z = tc_final(a4_0[:NN], a4_1[:NN], s4_0, s4_1, gat2_b, out_W, out_b)
    return (h, z)


# speedup vs baseline: 1.0215x; 1.0215x over previous
import jax
import jax.numpy as jnp
from jax import lax
from jax.experimental import pallas as pl
from jax.experimental.pallas import tpu as pltpu
from jax.experimental.pallas import tpu_sc as plsc

NN = 10000
NP2 = 10240
DD = 128
CC = 64
NEG_SLOPE = 0.2

ER = 330000
NC, NS, LANES = 2, 16, 16
NW = NC * NS
KE = 128
NCH = 82
BW = NCH * KE
EPAD = BW * NW
NPAIR = NCH // 2
RPS = NP2 // NS

_sc_mesh = plsc.VectorSubcoreMesh(core_axis_name="c", subcore_axis_name="s")
_sc_params = pltpu.CompilerParams(needs_layout_passes=False)


def _mm_pre_kernel(x_ref, w_ref, dis_ref, o_ref):
    xw = jnp.dot(x_ref[...], w_ref[...], preferred_element_type=jnp.float32)
    o_ref[...] = xw * dis_ref[...][:, None]


def tc_matmul_pre(x, w, dis):
    return pl.pallas_call(
        _mm_pre_kernel,
        out_shape=jax.ShapeDtypeStruct((x.shape[0], w.shape[1]), jnp.float32),
    )(x, w, dis)


def _dis_kernel(d0_ref, d1_ref, o_ref):
    d = d0_ref[...][:NN] + d1_ref[...][:NN]
    o_ref[...] = jnp.where(d > 0, lax.rsqrt(d), 0.0)


def tc_dis(deg0, deg1):
    return pl.pallas_call(
        _dis_kernel,
        out_shape=jax.ShapeDtypeStruct((NN,), jnp.float32),
    )(deg0, deg1)


def _post_gcn_prep_kernel(a0_ref, a1_ref, dis_ref, b_ref, w_ref, o_ref):
    acc = (a0_ref[...] + a1_ref[...]) * dis_ref[...][:, None]
    h = jnp.maximum(acc + b_ref[...][None, :], 0.0)
    o_ref[...] = jnp.dot(h, w_ref[...], preferred_element_type=jnp.float32)


def tc_post_gcn_prep(acc0, acc1, dis, b, w):
    return pl.pallas_call(
        _post_gcn_prep_kernel,
        out_shape=jax.ShapeDtypeStruct((NN, w.shape[1]), jnp.float32),
    )(acc0, acc1, dis, b, w)


def _att_kernel(xw_ref, asrc_ref, adst_ref, as_ref, ad_ref, u_ref):
    xw = xw_ref[...]
    a_s = jnp.sum(xw * asrc_ref[...][None, :], axis=1)
    a_d = jnp.sum(xw * adst_ref[...][None, :], axis=1)
    as_ref[...] = a_s
    ad_ref[...] = a_d
    e_ub = jnp.max(a_s) + a_d
    u_ref[...] = jnp.where(e_ub >= 0, e_ub, NEG_SLOPE * e_ub)


def tc_att(xw, a_src, a_dst):
    return pl.pallas_call(
        _att_kernel,
        out_shape=[jax.ShapeDtypeStruct((NN,), jnp.float32)] * 3,
    )(xw, a_src, a_dst)


def _post_gat_prep_kernel(a0_ref, a1_ref, s0_ref, s1_ref, b_ref, w_ref,
                          dis_ref, o_ref):
    s = (s0_ref[...][:NN] + s1_ref[...][:NN])[:, None] + 1e-16
    h = jnp.maximum((a0_ref[...] + a1_ref[...]) / s + b_ref[...][None, :], 0.0)
    xw = jnp.dot(h, w_ref[...], preferred_element_type=jnp.float32)
    o_ref[...] = xw * dis_ref[...][:, None]


def tc_post_gat_prep(acc0, acc1, s0, s1, b, w, dis):
    return pl.pallas_call(
        _post_gat_prep_kernel,
        out_shape=jax.ShapeDtypeStruct((NN, w.shape[1]), jnp.float32),
    )(acc0, acc1, s0, s1, b, w, dis)


def _final_kernel(a0_ref, a1_ref, s0_ref, s1_ref, b_ref, w_ref, ob_ref,
                  h_ref, z_ref):
    s = (s0_ref[...][:NN] + s1_ref[...][:NN])[:, None] + 1e-16
    h = jnp.maximum((a0_ref[...] + a1_ref[...]) / s + b_ref[...][None, :], 0.0)
    h_ref[...] = h
    z_ref[...] = (jnp.dot(h, w_ref[...], preferred_element_type=jnp.float32)
                  + ob_ref[...][None, :])


def tc_final(acc0, acc1, s0, s1, b, w, ob):
    return pl.pallas_call(
        _final_kernel,
        out_shape=[jax.ShapeDtypeStruct((NN, DD), jnp.float32),
                   jax.ShapeDtypeStruct((NN, CC), jnp.float32)],
    )(acc0, acc1, s0, s1, b, w, ob)


def _edge_mask(eb, j):
    ids = eb + j * LANES + lax.iota(jnp.int32, LANES)
    return ids < ER


def _deg_body(dst_hbm, z1_hbm, out0, out1, didx, ones_v, deg_sh, sem):
    c = lax.axis_index("c")
    s = lax.axis_index("s")
    wid = c * NS + s
    pltpu.sync_copy(z1_hbm, deg_sh.at[pl.ds(s * RPS, RPS)])
    for j in range(KE // LANES):
        ones_v[pl.ds(j * LANES, LANES)] = jnp.full((LANES,), 1.0, jnp.float32)
    plsc.subcore_barrier()

    def chunk(i, carry):
        eb = (i * NW + wid) * KE
        pltpu.sync_copy(dst_hbm.at[pl.ds(eb, KE)], didx)
        pltpu.sync_copy(ones_v, deg_sh.at[didx], add=True)
        return carry

    lax.fori_loop(0, NCH, chunk, 0)
    plsc.subcore_barrier()
    sl = pl.ds(s * RPS, RPS)

    @pl.when(c == 0)
    def _():
        pltpu.sync_copy(deg_sh.at[sl], out0.at[sl])

    @pl.when(c == 1)
    def _():
        pltpu.sync_copy(deg_sh.at[sl], out1.at[sl])


def sc_degree(dst_p, z1):
    f = pl.kernel(
        _deg_body,
        out_type=[jax.ShapeDtypeStruct((NP2,), jnp.float32)] * 2,
        mesh=_sc_mesh,
        compiler_params=_sc_params,
        scratch_types=[
            pltpu.VMEM((KE,), jnp.int32),
            pltpu.VMEM((KE,), jnp.float32),
            pltpu.VMEM_SHARED((NP2,), jnp.float32),
            pltpu.SemaphoreType.DMA,
        ],
    )
    return f(dst_p, z1)


def _gcn_body(src_hbm, dst_hbm, xw_hbm, z2_hbm, out0, out1,
              sidx, didx, rows, acc_sh, sem):
    c = lax.axis_index("c")
    s = lax.axis_index("s")
    wid = c * NS + s
    pltpu.sync_copy(z2_hbm, acc_sh.at[pl.ds(s * RPS, RPS)])
    plsc.subcore_barrier()

    def chunk(i, carry):
        eb = (i * NW + wid) * KE
        pltpu.sync_copy(src_hbm.at[pl.ds(eb, KE)], sidx)
        pltpu.sync_copy(dst_hbm.at[pl.ds(eb, KE)], didx)
        pltpu.async_copy(xw_hbm.at[sidx], rows, sem).wait()
        pltpu.sync_copy(rows, acc_sh.at[didx], add=True)
        return carry

    lax.fori_loop(0, NCH, chunk, 0)
    plsc.subcore_barrier()
    sl = pl.ds(s * RPS, RPS)

    @pl.when(c == 0)
    def _():
        pltpu.sync_copy(acc_sh.at[sl], out0.at[sl])

    @pl.when(c == 1)
    def _():
        pltpu.sync_copy(acc_sh.at[sl], out1.at[sl])


def sc_gcn_pass(src_p, dst_p, xw, z2):
    f = pl.kernel(
        _gcn_body,
        out_type=[jax.ShapeDtypeStruct((NP2, DD), jnp.float32)] * 2,
        mesh=_sc_mesh,
        compiler_params=_sc_params,
        scratch_types=[
            pltpu.VMEM((KE,), jnp.int32),
            pltpu.VMEM((KE,), jnp.int32),
            pltpu.VMEM((KE, DD), jnp.float32),
            pltpu.VMEM_SHARED((NP2, DD), jnp.float32),
            pltpu.SemaphoreType.DMA,
        ],
    )
    return f(src_p, dst_p, xw, z2)


def _gat_body(src_hbm, dst_hbm, as_hbm, ad_hbm, u_hbm, xw_hbm, z2_hbm, z1_hbm,
              out0, out1, so0, so1,
              as_v, ad_v, u_v, sidx, didx, rows, wv, acc_sh, s_sh, sem):
    c = lax.axis_index("c")
    s = lax.axis_index("s")
    wid = c * NS + s
    pltpu.sync_copy(z2_hbm, acc_sh.at[pl.ds(s * RPS, RPS)])
    pltpu.sync_copy(z1_hbm, s_sh.at[pl.ds(s * RPS, RPS)])
    pltpu.sync_copy(as_hbm, as_v)
    pltpu.sync_copy(ad_hbm, ad_v)
    pltpu.sync_copy(u_hbm, u_v)
    plsc.subcore_barrier()

    def chunk(i, carry):
        eb = (i * NW + wid) * KE
        pltpu.sync_copy(src_hbm.at[pl.ds(eb, KE)], sidx)
        pltpu.sync_copy(dst_hbm.at[pl.ds(eb, KE)], didx)
        pltpu.async_copy(xw_hbm.at[sidx], rows, sem).wait()

        def grp(j, carry2):
            sl = pl.ds(j * LANES, LANES)
            sv = sidx[sl]
            dv = didx[sl]
            e = plsc.load_gather(as_v, [sv]) + plsc.load_gather(ad_v, [dv])
            e = jnp.where(e >= 0, e, NEG_SLOPE * e)
            ex = jnp.exp(e - plsc.load_gather(u_v, [dv]))
            ex = jnp.where(_edge_mask(eb, j), ex, 0.0)
            wv[sl] = ex
            return carry2

        lax.fori_loop(0, KE // LANES, grp, 0)

        def scale(e, carry2):
            w = plsc.load_gather(wv, [jnp.full((LANES,), e, jnp.int32)])
            for jj in range(DD // LANES):
                sl = pl.ds(jj * LANES, LANES)
                rows[e, sl] = rows[e, sl] * w
            return carry2

        lax.fori_loop(0, KE, scale, 0, unroll=2)
        pltpu.sync_copy(rows, acc_sh.at[didx], add=True)
        pltpu.sync_copy(wv, s_sh.at[didx], add=True)
        return carry

    lax.fori_loop(0, NCH, chunk, 0)
    plsc.subcore_barrier()
    sl = pl.ds(s * RPS, RPS)

    @pl.when(c == 0)
    def _():
        pltpu.sync_copy(acc_sh.at[sl], out0.at[sl])
        pltpu.sync_copy(s_sh.at[sl], so0.at[sl])

    @pl.when(c == 1)
    def _():
        pltpu.sync_copy(acc_sh.at[sl], out1.at[sl])
        pltpu.sync_copy(s_sh.at[sl], so1.at[sl])


def sc_gat_pass(src_p, dst_p, a_s, a_d, u, xw, z2, z1):
    f = pl.kernel(
        _gat_body,
        out_type=[jax.ShapeDtypeStruct((NP2, DD), jnp.float32)] * 2
        + [jax.ShapeDtypeStruct((NP2,), jnp.float32)] * 2,
        mesh=_sc_mesh,
        compiler_params=_sc_params,
        scratch_types=[
            pltpu.VMEM((NN,), jnp.float32),
            pltpu.VMEM((NN,), jnp.float32),
            pltpu.VMEM((NN,), jnp.float32),
            pltpu.VMEM((KE,), jnp.int32),
            pltpu.VMEM((KE,), jnp.int32),
            pltpu.VMEM((KE, DD), jnp.float32),
            pltpu.VMEM((KE,), jnp.float32),
            pltpu.VMEM_SHARED((NP2, DD), jnp.float32),
            pltpu.VMEM_SHARED((NP2,), jnp.float32),
            pltpu.SemaphoreType.DMA,
        ],
    )
    return f(src_p, dst_p, a_s, a_d, u, xw, z2, z1)


def kernel(x, edge_index, gcn1_W, gcn1_b, gat1_W, gat1_att_src, gat1_att_dst,
           gat1_b, gcn2_W, gcn2_b, gat2_W, gat2_att_src, gat2_att_dst,
           gat2_b, out_W, out_b):
    loops = jnp.arange(NN, dtype=jnp.int32)
    spad = jnp.zeros((EPAD - ER,), jnp.int32)
    dpad = NN + (jnp.arange(EPAD - ER, dtype=jnp.int32) % (NP2 - NN))
    src_p = jnp.concatenate([edge_index[0].astype(jnp.int32), loops, spad])
    dst_p = jnp.concatenate([edge_index[1].astype(jnp.int32), loops, dpad])
    z1 = jnp.zeros((RPS,), jnp.float32)
    z2 = jnp.zeros((RPS, DD), jnp.float32)

    deg0, deg1 = sc_degree(dst_p, z1)
    dis = tc_dis(deg0, deg1)

    xw1 = tc_matmul_pre(x, gcn1_W, dis)
    a1_0, a1_1 = sc_gcn_pass(src_p, dst_p, xw1, z2)

    xw2 = tc_post_gcn_prep(a1_0[:NN], a1_1[:NN], dis, gcn1_b, gat1_W)
    as2, ad2, u2 = tc_att(xw2, gat1_att_src, gat1_att_dst)
    a2_0, a2_1, s2_0, s2_1 = sc_gat_pass(src_p, dst_p, as2, ad2, u2, xw2,
                                         z2, z1)

    xw3 = tc_post_gat_prep(a2_0[:NN], a2_1[:NN], s2_0, s2_1, gat1_b, gcn2_W,
                           dis)
    a3_0, a3_1 = sc_gcn_pass(src_p, dst_p, xw3, z2)

    xw4 = tc_post_gcn_prep(a3_0[:NN], a3_1[:NN], dis, gcn2_b, gat2_W)
    as4, ad4, u4 = tc_att(xw4, gat2_att_src, gat2_att_dst)
    a4_0, a4_1, s4_0, s4_1 = sc_gat_pass(src_p, dst_p, as4, ad4, u4, xw4,
                                         z2, z1)

    h, z = tc_final(a4_0[:NN], a4_1[:NN], s4_0, s4_1, gat2_b, out_W, out_b)
    return (h, z)


# speedup vs baseline: 1.3345x; 1.3064x over previous
import jax
import jax.numpy as jnp
from jax import lax
from jax.experimental import pallas as pl
from jax.experimental.pallas import tpu as pltpu
from jax.experimental.pallas import tpu_sc as plsc

NN = 10000
NP2 = 10240
DD = 128
CC = 64
NEG_SLOPE = 0.2

ER = 330000
NC, NS, LANES = 2, 16, 16
NW = NC * NS
KE = 128
NCH = 81
BW = NCH * KE
EPAD = BW * NW
RPS = NP2 // NS

_sc_mesh = plsc.VectorSubcoreMesh(core_axis_name="c", subcore_axis_name="s")
_sc_params = pltpu.CompilerParams(needs_layout_passes=False)


def _mm_pre_kernel(x_ref, w_ref, dis_ref, o_ref):
    xw = jnp.dot(x_ref[...], w_ref[...], preferred_element_type=jnp.float32)
    o_ref[...] = xw * dis_ref[...][:, None]


def tc_matmul_pre(x, w, dis):
    return pl.pallas_call(
        _mm_pre_kernel,
        out_shape=jax.ShapeDtypeStruct((x.shape[0], w.shape[1]), jnp.float32),
    )(x, w, dis)


def _dis_kernel(d0_ref, d1_ref, o_ref):
    d = d0_ref[...][:NN] + d1_ref[...][:NN]
    o_ref[...] = jnp.where(d > 0, lax.rsqrt(d), 0.0)


def tc_dis(deg0, deg1):
    return pl.pallas_call(
        _dis_kernel,
        out_shape=jax.ShapeDtypeStruct((NN,), jnp.float32),
    )(deg0, deg1)


def _post_gcn_prep_kernel(a0_ref, a1_ref, dis_ref, b_ref, w_ref, o_ref):
    acc = (a0_ref[...] + a1_ref[...]) * dis_ref[...][:, None]
    h = jnp.maximum(acc + b_ref[...][None, :], 0.0)
    o_ref[...] = jnp.dot(h, w_ref[...], preferred_element_type=jnp.float32)


def tc_post_gcn_prep(acc0, acc1, dis, b, w):
    return pl.pallas_call(
        _post_gcn_prep_kernel,
        out_shape=jax.ShapeDtypeStruct((NN, w.shape[1]), jnp.float32),
    )(acc0, acc1, dis, b, w)


def _att_kernel(xw_ref, asrc_ref, adst_ref, as_ref, ad_ref, u_ref):
    xw = xw_ref[...]
    a_s = jnp.sum(xw * asrc_ref[...][None, :], axis=1)
    a_d = jnp.sum(xw * adst_ref[...][None, :], axis=1)
    as_ref[...] = a_s
    ad_ref[...] = a_d
    e_ub = jnp.max(a_s) + a_d
    u_ref[...] = jnp.where(e_ub >= 0, e_ub, NEG_SLOPE * e_ub)


def tc_att(xw, a_src, a_dst):
    return pl.pallas_call(
        _att_kernel,
        out_shape=[jax.ShapeDtypeStruct((NN,), jnp.float32)] * 3,
    )(xw, a_src, a_dst)


def _post_gat_prep_kernel(a0_ref, a1_ref, s0_ref, s1_ref, b_ref, w_ref,
                          dis_ref, o_ref):
    s = (s0_ref[...][:NN] + s1_ref[...][:NN])[:, None] + 1e-16
    h = jnp.maximum((a0_ref[...] + a1_ref[...]) / s + b_ref[...][None, :], 0.0)
    xw = jnp.dot(h, w_ref[...], preferred_element_type=jnp.float32)
    o_ref[...] = xw * dis_ref[...][:, None]


def tc_post_gat_prep(acc0, acc1, s0, s1, b, w, dis):
    return pl.pallas_call(
        _post_gat_prep_kernel,
        out_shape=jax.ShapeDtypeStruct((NN, w.shape[1]), jnp.float32),
    )(acc0, acc1, s0, s1, b, w, dis)


def _final_kernel(a0_ref, a1_ref, s0_ref, s1_ref, b_ref, w_ref, ob_ref,
                  h_ref, z_ref):
    s = (s0_ref[...][:NN] + s1_ref[...][:NN])[:, None] + 1e-16
    h = jnp.maximum((a0_ref[...] + a1_ref[...]) / s + b_ref[...][None, :], 0.0)
    h_ref[...] = h
    z_ref[...] = (jnp.dot(h, w_ref[...], preferred_element_type=jnp.float32)
                  + ob_ref[...][None, :])


def tc_final(acc0, acc1, s0, s1, b, w, ob):
    return pl.pallas_call(
        _final_kernel,
        out_shape=[jax.ShapeDtypeStruct((NN, DD), jnp.float32),
                   jax.ShapeDtypeStruct((NN, CC), jnp.float32)],
    )(acc0, acc1, s0, s1, b, w, ob)


def _edge_mask(eb, j):
    ids = eb + j * LANES + lax.iota(jnp.int32, LANES)
    return ids < ER


def _deg_body(dst_hbm, z1_hbm, out0, out1, didx, ones_v, deg_sh, sem):
    c = lax.axis_index("c")
    s = lax.axis_index("s")
    wid = c * NS + s
    pltpu.sync_copy(z1_hbm, deg_sh.at[pl.ds(s * RPS, RPS)])
    for j in range(KE // LANES):
        ones_v[pl.ds(j * LANES, LANES)] = jnp.full((LANES,), 1.0, jnp.float32)
    plsc.subcore_barrier()

    def chunk(i, carry):
        eb = wid * BW + i * KE
        pltpu.sync_copy(dst_hbm.at[pl.ds(eb, KE)], didx)
        pltpu.sync_copy(ones_v, deg_sh.at[didx], add=True)
        return carry

    lax.fori_loop(0, NCH, chunk, 0)
    plsc.subcore_barrier()
    sl = pl.ds(s * RPS, RPS)

    @pl.when(c == 0)
    def _():
        pltpu.sync_copy(deg_sh.at[sl], out0.at[sl])

    @pl.when(c == 1)
    def _():
        pltpu.sync_copy(deg_sh.at[sl], out1.at[sl])


def sc_degree(dst_p, z1):
    f = pl.kernel(
        _deg_body,
        out_type=[jax.ShapeDtypeStruct((NP2,), jnp.float32)] * 2,
        mesh=_sc_mesh,
        compiler_params=_sc_params,
        scratch_types=[
            pltpu.VMEM((KE,), jnp.int32),
            pltpu.VMEM((KE,), jnp.float32),
            pltpu.VMEM_SHARED((NP2,), jnp.float32),
            pltpu.SemaphoreType.DMA,
        ],
    )
    return f(dst_p, z1)


def _gcn_body(src_hbm, dst_hbm, xw_hbm, z2_hbm, out0, out1,
              sidx, didx, rows, acc_sh, sem):
    c = lax.axis_index("c")
    s = lax.axis_index("s")
    wid = c * NS + s
    pltpu.sync_copy(z2_hbm, acc_sh.at[pl.ds(s * RPS, RPS)])
    plsc.subcore_barrier()

    def chunk(i, carry):
        eb = wid * BW + i * KE
        pltpu.sync_copy(src_hbm.at[pl.ds(eb, KE)], sidx)
        pltpu.sync_copy(dst_hbm.at[pl.ds(eb, KE)], didx)
        pltpu.async_copy(xw_hbm.at[sidx], rows, sem).wait()
        pltpu.sync_copy(rows, acc_sh.at[didx], add=True)
        return carry

    lax.fori_loop(0, NCH, chunk, 0)
    plsc.subcore_barrier()
    sl = pl.ds(s * RPS, RPS)

    @pl.when(c == 0)
    def _():
        pltpu.sync_copy(acc_sh.at[sl], out0.at[sl])

    @pl.when(c == 1)
    def _():
        pltpu.sync_copy(acc_sh.at[sl], out1.at[sl])


def sc_gcn_pass(src_p, dst_p, xw, z2):
    f = pl.kernel(
        _gcn_body,
        out_type=[jax.ShapeDtypeStruct((NP2, DD), jnp.float32)] * 2,
        mesh=_sc_mesh,
        compiler_params=_sc_params,
        scratch_types=[
            pltpu.VMEM((KE,), jnp.int32),
            pltpu.VMEM((KE,), jnp.int32),
            pltpu.VMEM((KE, DD), jnp.float32),
            pltpu.VMEM_SHARED((NP2, DD), jnp.float32),
            pltpu.SemaphoreType.DMA,
        ],
    )
    return f(src_p, dst_p, xw, z2)


def _gat_body(src_hbm, dst_hbm, as_hbm, ad_hbm, u_hbm, xw_hbm, z2_hbm, z1_hbm,
              out0, out1, so0, so1,
              as_v, ad_v, u_v, sidx, didx, rows, wv, acc_sh, s_sh, sem):
    c = lax.axis_index("c")
    s = lax.axis_index("s")
    wid = c * NS + s
    pltpu.sync_copy(z2_hbm, acc_sh.at[pl.ds(s * RPS, RPS)])
    pltpu.sync_copy(z1_hbm, s_sh.at[pl.ds(s * RPS, RPS)])
    pltpu.sync_copy(as_hbm, as_v)
    pltpu.sync_copy(ad_hbm, ad_v)
    pltpu.sync_copy(u_hbm, u_v)
    plsc.subcore_barrier()

    def chunk(i, carry):
        eb = wid * BW + i * KE
        pltpu.sync_copy(src_hbm.at[pl.ds(eb, KE)], sidx)
        pltpu.sync_copy(dst_hbm.at[pl.ds(eb, KE)], didx)
        pltpu.async_copy(xw_hbm.at[sidx], rows, sem).wait()

        def grp(j, carry2):
            sl = pl.ds(j * LANES, LANES)
            sv = sidx[sl]
            dv = didx[sl]
            e = plsc.load_gather(as_v, [sv]) + plsc.load_gather(ad_v, [dv])
            e = jnp.where(e >= 0, e, NEG_SLOPE * e)
            ex = jnp.exp(e - plsc.load_gather(u_v, [dv]))
            ex = jnp.where(_edge_mask(eb, j), ex, 0.0)
            wv[sl] = ex
            return carry2

        lax.fori_loop(0, KE // LANES, grp, 0)

        def scale(e, carry2):
            w = plsc.load_gather(wv, [jnp.full((LANES,), e, jnp.int32)])
            for jj in range(DD // LANES):
                sl = pl.ds(jj * LANES, LANES)
                rows[e, sl] = rows[e, sl] * w
            return carry2

        lax.fori_loop(0, KE, scale, 0)
        pltpu.sync_copy(rows, acc_sh.at[didx], add=True)
        pltpu.sync_copy(wv, s_sh.at[didx], add=True)
        return carry

    lax.fori_loop(0, NCH, chunk, 0)
    plsc.subcore_barrier()
    sl = pl.ds(s * RPS, RPS)

    @pl.when(c == 0)
    def _():
        pltpu.sync_copy(acc_sh.at[sl], out0.at[sl])
        pltpu.sync_copy(s_sh.at[sl], so0.at[sl])

    @pl.when(c == 1)
    def _():
        pltpu.sync_copy(acc_sh.at[sl], out1.at[sl])
        pltpu.sync_copy(s_sh.at[sl], so1.at[sl])


def sc_gat_pass(src_p, dst_p, a_s, a_d, u, xw, z2, z1):
    f = pl.kernel(
        _gat_body,
        out_type=[jax.ShapeDtypeStruct((NP2, DD), jnp.float32)] * 2
        + [jax.ShapeDtypeStruct((NP2,), jnp.float32)] * 2,
        mesh=_sc_mesh,
        compiler_params=_sc_params,
        scratch_types=[
            pltpu.VMEM((NN,), jnp.float32),
            pltpu.VMEM((NN,), jnp.float32),
            pltpu.VMEM((NN,), jnp.float32),
            pltpu.VMEM((KE,), jnp.int32),
            pltpu.VMEM((KE,), jnp.int32),
            pltpu.VMEM((KE, DD), jnp.float32),
            pltpu.VMEM((KE,), jnp.float32),
            pltpu.VMEM_SHARED((NP2, DD), jnp.float32),
            pltpu.VMEM_SHARED((NP2,), jnp.float32),
            pltpu.SemaphoreType.DMA,
        ],
    )
    return f(src_p, dst_p, a_s, a_d, u, xw, z2, z1)


def kernel(x, edge_index, gcn1_W, gcn1_b, gat1_W, gat1_att_src, gat1_att_dst,
           gat1_b, gcn2_W, gcn2_b, gat2_W, gat2_att_src, gat2_att_dst,
           gat2_b, out_W, out_b):
    loops = jnp.arange(NN, dtype=jnp.int32)
    spad = jnp.zeros((EPAD - ER,), jnp.int32)
    dpad = NN + (jnp.arange(EPAD - ER, dtype=jnp.int32) % (NP2 - NN))
    src_p = jnp.concatenate([edge_index[0].astype(jnp.int32), loops, spad])
    dst_p = jnp.concatenate([edge_index[1].astype(jnp.int32), loops, dpad])
    z1 = jnp.zeros((RPS,), jnp.float32)
    z2 = jnp.zeros((RPS, DD), jnp.float32)

    deg0, deg1 = sc_degree(dst_p, z1)
    dis = tc_dis(deg0, deg1)

    xw1 = tc_matmul_pre(x, gcn1_W, dis)
    a1_0, a1_1 = sc_gcn_pass(src_p, dst_p, xw1, z2)

    xw2 = tc_post_gcn_prep(a1_0[:NN], a1_1[:NN], dis, gcn1_b, gat1_W)
    as2, ad2, u2 = tc_att(xw2, gat1_att_src, gat1_att_dst)
    a2_0, a2_1, s2_0, s2_1 = sc_gat_pass(src_p, dst_p, as2, ad2, u2, xw2,
                                         z2, z1)

    xw3 = tc_post_gat_prep(a2_0[:NN], a2_1[:NN], s2_0, s2_1, gat1_b, gcn2_W,
                           dis)
    a3_0, a3_1 = sc_gcn_pass(src_p, dst_p, xw3, z2)

    xw4 = tc_post_gcn_prep(a3_0[:NN], a3_1[:NN], dis, gcn2_b, gat2_W)
    as4, ad4, u4 = tc_att(xw4, gat2_att_src, gat2_att_dst)
    a4_0, a4_1, s4_0, s4_1 = sc_gat_pass(src_p, dst_p, as4, ad4, u4, xw4,
                                         z2, z1)

    h, z = tc_final(a4_0[:NN], a4_1[:NN], s4_0, s4_1, gat2_b, out_W, out_b)
    return (h, z)


# speedup vs baseline: 1.4783x; 1.1078x over previous
import jax
import jax.numpy as jnp
from jax import lax
from jax.experimental import pallas as pl
from jax.experimental.pallas import tpu as pltpu
from jax.experimental.pallas import tpu_sc as plsc

NN = 10000
NP2 = 10240
DD = 128
CC = 64
NEG_SLOPE = 0.2

ER = 330000
NC, NS, LANES = 2, 16, 16
NW = NC * NS
KE = 128
NCH = 81
BW = NCH * KE
EPAD = BW * NW
RPS = NP2 // NS

_sc_mesh = plsc.VectorSubcoreMesh(core_axis_name="c", subcore_axis_name="s")
_sc_params = pltpu.CompilerParams(needs_layout_passes=False)


def _mm_pre_kernel(x_ref, w_ref, dis_ref, o_ref):
    xw = jnp.dot(x_ref[...], w_ref[...], preferred_element_type=jnp.float32)
    o_ref[...] = xw * dis_ref[...][:, None]


def tc_matmul_pre(x, w, dis):
    return pl.pallas_call(
        _mm_pre_kernel,
        out_shape=jax.ShapeDtypeStruct((x.shape[0], w.shape[1]), jnp.float32),
    )(x, w, dis)


def _dis_kernel(d0_ref, d1_ref, o_ref):
    d = d0_ref[...][:NN] + d1_ref[...][:NN]
    o_ref[...] = jnp.where(d > 0, lax.rsqrt(d), 0.0)


def tc_dis(deg0, deg1):
    return pl.pallas_call(
        _dis_kernel,
        out_shape=jax.ShapeDtypeStruct((NN,), jnp.float32),
    )(deg0, deg1)


def _post_gcn_prep_kernel(a0_ref, a1_ref, dis_ref, b_ref, w_ref, o_ref):
    acc = (a0_ref[...] + a1_ref[...]) * dis_ref[...][:, None]
    h = jnp.maximum(acc + b_ref[...][None, :], 0.0)
    o_ref[...] = jnp.dot(h, w_ref[...], preferred_element_type=jnp.float32)


def tc_post_gcn_prep(acc0, acc1, dis, b, w):
    return pl.pallas_call(
        _post_gcn_prep_kernel,
        out_shape=jax.ShapeDtypeStruct((NN, w.shape[1]), jnp.float32),
    )(acc0, acc1, dis, b, w)


def _att_kernel(xw_ref, asrc_ref, adst_ref, as_ref, ad_ref, u_ref):
    xw = xw_ref[...]
    a_s = jnp.sum(xw * asrc_ref[...][None, :], axis=1)
    a_d = jnp.sum(xw * adst_ref[...][None, :], axis=1)
    as_ref[...] = a_s
    ad_ref[...] = a_d
    e_ub = jnp.max(a_s) + a_d
    u_ref[...] = jnp.where(e_ub >= 0, e_ub, NEG_SLOPE * e_ub)


def tc_att(xw, a_src, a_dst):
    return pl.pallas_call(
        _att_kernel,
        out_shape=[jax.ShapeDtypeStruct((NN,), jnp.float32)] * 3,
    )(xw, a_src, a_dst)


def _post_gat_prep_kernel(a0_ref, a1_ref, s0_ref, s1_ref, b_ref, w_ref,
                          dis_ref, o_ref):
    s = (s0_ref[...][:NN] + s1_ref[...][:NN])[:, None] + 1e-16
    h = jnp.maximum((a0_ref[...] + a1_ref[...]) / s + b_ref[...][None, :], 0.0)
    xw = jnp.dot(h, w_ref[...], preferred_element_type=jnp.float32)
    o_ref[...] = xw * dis_ref[...][:, None]


def tc_post_gat_prep(acc0, acc1, s0, s1, b, w, dis):
    return pl.pallas_call(
        _post_gat_prep_kernel,
        out_shape=jax.ShapeDtypeStruct((NN, w.shape[1]), jnp.float32),
    )(acc0, acc1, s0, s1, b, w, dis)


def _final_kernel(a0_ref, a1_ref, s0_ref, s1_ref, b_ref, w_ref, ob_ref,
                  h_ref, z_ref):
    s = (s0_ref[...][:NN] + s1_ref[...][:NN])[:, None] + 1e-16
    h = jnp.maximum((a0_ref[...] + a1_ref[...]) / s + b_ref[...][None, :], 0.0)
    h_ref[...] = h
    z_ref[...] = (jnp.dot(h, w_ref[...], preferred_element_type=jnp.float32)
                  + ob_ref[...][None, :])


def tc_final(acc0, acc1, s0, s1, b, w, ob):
    return pl.pallas_call(
        _final_kernel,
        out_shape=[jax.ShapeDtypeStruct((NN, DD), jnp.float32),
                   jax.ShapeDtypeStruct((NN, CC), jnp.float32)],
    )(acc0, acc1, s0, s1, b, w, ob)


def _edge_mask(eb, j):
    ids = eb + j * LANES + lax.iota(jnp.int32, LANES)
    return ids < ER


def _deg_body(dst_hbm, z1_hbm, out0, out1, didx, ones_v, deg_sh, sem):
    c = lax.axis_index("c")
    s = lax.axis_index("s")
    wid = c * NS + s
    pltpu.sync_copy(z1_hbm, deg_sh.at[pl.ds(s * RPS, RPS)])
    for j in range(KE // LANES):
        ones_v[pl.ds(j * LANES, LANES)] = jnp.full((LANES,), 1.0, jnp.float32)
    plsc.subcore_barrier()

    def chunk(i, carry):
        eb = wid * BW + i * KE
        pltpu.sync_copy(dst_hbm.at[pl.ds(eb, KE)], didx)
        pltpu.sync_copy(ones_v, deg_sh.at[didx], add=True)
        return carry

    lax.fori_loop(0, NCH, chunk, 0)
    plsc.subcore_barrier()
    sl = pl.ds(s * RPS, RPS)

    @pl.when(c == 0)
    def _():
        pltpu.sync_copy(deg_sh.at[sl], out0.at[sl])

    @pl.when(c == 1)
    def _():
        pltpu.sync_copy(deg_sh.at[sl], out1.at[sl])


def sc_degree(dst_p, z1):
    f = pl.kernel(
        _deg_body,
        out_type=[jax.ShapeDtypeStruct((NP2,), jnp.float32)] * 2,
        mesh=_sc_mesh,
        compiler_params=_sc_params,
        scratch_types=[
            pltpu.VMEM((KE,), jnp.int32),
            pltpu.VMEM((KE,), jnp.float32),
            pltpu.VMEM_SHARED((NP2,), jnp.float32),
            pltpu.SemaphoreType.DMA,
        ],
    )
    return f(dst_p, z1)


def _gcn_body(src_hbm, dst_hbm, xw_hbm, z2_hbm, out0, out1,
              sidx, didx, rows, acc_sh, sem):
    c = lax.axis_index("c")
    s = lax.axis_index("s")
    wid = c * NS + s
    pltpu.sync_copy(z2_hbm, acc_sh.at[pl.ds(s * RPS, RPS)])
    plsc.subcore_barrier()

    def chunk(i, carry):
        eb = wid * BW + i * KE
        pltpu.sync_copy(src_hbm.at[pl.ds(eb, KE)], sidx)
        g = pltpu.async_copy(xw_hbm.at[sidx], rows, sem)
        pltpu.sync_copy(dst_hbm.at[pl.ds(eb, KE)], didx)
        g.wait()
        pltpu.sync_copy(rows, acc_sh.at[didx], add=True)
        return carry

    lax.fori_loop(0, NCH, chunk, 0)
    plsc.subcore_barrier()
    sl = pl.ds(s * RPS, RPS)

    @pl.when(c == 0)
    def _():
        pltpu.sync_copy(acc_sh.at[sl], out0.at[sl])

    @pl.when(c == 1)
    def _():
        pltpu.sync_copy(acc_sh.at[sl], out1.at[sl])


def sc_gcn_pass(src_p, dst_p, xw, z2):
    f = pl.kernel(
        _gcn_body,
        out_type=[jax.ShapeDtypeStruct((NP2, DD), jnp.float32)] * 2,
        mesh=_sc_mesh,
        compiler_params=_sc_params,
        scratch_types=[
            pltpu.VMEM((KE,), jnp.int32),
            pltpu.VMEM((KE,), jnp.int32),
            pltpu.VMEM((KE, DD), jnp.float32),
            pltpu.VMEM_SHARED((NP2, DD), jnp.float32),
            pltpu.SemaphoreType.DMA,
        ],
    )
    return f(src_p, dst_p, xw, z2)


def _gat_body(src_hbm, dst_hbm, as_hbm, ad_hbm, u_hbm, xw_hbm, z2_hbm, z1_hbm,
              out0, out1, so0, so1,
              as_v, ad_v, u_v, sidx, didx, rows, wv, acc_sh, s_sh, sem):
    c = lax.axis_index("c")
    s = lax.axis_index("s")
    wid = c * NS + s
    pltpu.sync_copy(z2_hbm, acc_sh.at[pl.ds(s * RPS, RPS)])
    pltpu.sync_copy(z1_hbm, s_sh.at[pl.ds(s * RPS, RPS)])
    pltpu.sync_copy(as_hbm, as_v)
    pltpu.sync_copy(ad_hbm, ad_v)
    pltpu.sync_copy(u_hbm, u_v)
    plsc.subcore_barrier()

    def chunk(i, carry):
        eb = wid * BW + i * KE
        pltpu.sync_copy(src_hbm.at[pl.ds(eb, KE)], sidx)
        g = pltpu.async_copy(xw_hbm.at[sidx], rows, sem)
        pltpu.sync_copy(dst_hbm.at[pl.ds(eb, KE)], didx)

        def grp(j, carry2):
            sl = pl.ds(j * LANES, LANES)
            sv = sidx[sl]
            dv = didx[sl]
            e = plsc.load_gather(as_v, [sv]) + plsc.load_gather(ad_v, [dv])
            e = jnp.where(e >= 0, e, NEG_SLOPE * e)
            ex = jnp.exp(e - plsc.load_gather(u_v, [dv]))
            ex = jnp.where(_edge_mask(eb, j), ex, 0.0)
            wv[sl] = ex
            return carry2

        lax.fori_loop(0, KE // LANES, grp, 0)
        g.wait()

        def scale(e, carry2):
            w = plsc.load_gather(wv, [jnp.full((LANES,), e, jnp.int32)])
            for jj in range(DD // LANES):
                sl = pl.ds(jj * LANES, LANES)
                rows[e, sl] = rows[e, sl] * w
            return carry2

        lax.fori_loop(0, KE, scale, 0)
        pltpu.sync_copy(rows, acc_sh.at[didx], add=True)
        pltpu.sync_copy(wv, s_sh.at[didx], add=True)
        return carry

    lax.fori_loop(0, NCH, chunk, 0)
    plsc.subcore_barrier()
    sl = pl.ds(s * RPS, RPS)

    @pl.when(c == 0)
    def _():
        pltpu.sync_copy(acc_sh.at[sl], out0.at[sl])
        pltpu.sync_copy(s_sh.at[sl], so0.at[sl])

    @pl.when(c == 1)
    def _():
        pltpu.sync_copy(acc_sh.at[sl], out1.at[sl])
        pltpu.sync_copy(s_sh.at[sl], so1.at[sl])


def sc_gat_pass(src_p, dst_p, a_s, a_d, u, xw, z2, z1):
    f = pl.kernel(
        _gat_body,
        out_type=[jax.ShapeDtypeStruct((NP2, DD), jnp.float32)] * 2
        + [jax.ShapeDtypeStruct((NP2,), jnp.float32)] * 2,
        mesh=_sc_mesh,
        compiler_params=_sc_params,
        scratch_types=[
            pltpu.VMEM((NN,), jnp.float32),
            pltpu.VMEM((NN,), jnp.float32),
            pltpu.VMEM((NN,), jnp.float32),
            pltpu.VMEM((KE,), jnp.int32),
            pltpu.VMEM((KE,), jnp.int32),
            pltpu.VMEM((KE, DD), jnp.float32),
            pltpu.VMEM((KE,), jnp.float32),
            pltpu.VMEM_SHARED((NP2, DD), jnp.float32),
            pltpu.VMEM_SHARED((NP2,), jnp.float32),
            pltpu.SemaphoreType.DMA,
        ],
    )
    return f(src_p, dst_p, a_s, a_d, u, xw, z2, z1)


def kernel(x, edge_index, gcn1_W, gcn1_b, gat1_W, gat1_att_src, gat1_att_dst,
           gat1_b, gcn2_W, gcn2_b, gat2_W, gat2_att_src, gat2_att_dst,
           gat2_b, out_W, out_b):
    loops = jnp.arange(NN, dtype=jnp.int32)
    spad = jnp.zeros((EPAD - ER,), jnp.int32)
    dpad = NN + (jnp.arange(EPAD - ER, dtype=jnp.int32) % (NP2 - NN))
    src_p = jnp.concatenate([edge_index[0].astype(jnp.int32), loops, spad])
    dst_p = jnp.concatenate([edge_index[1].astype(jnp.int32), loops, dpad])
    z1 = jnp.zeros((RPS,), jnp.float32)
    z2 = jnp.zeros((RPS, DD), jnp.float32)

    deg0, deg1 = sc_degree(dst_p, z1)
    dis = tc_dis(deg0, deg1)

    xw1 = tc_matmul_pre(x, gcn1_W, dis)
    a1_0, a1_1 = sc_gcn_pass(src_p, dst_p, xw1, z2)

    xw2 = tc_post_gcn_prep(a1_0[:NN], a1_1[:NN], dis, gcn1_b, gat1_W)
    as2, ad2, u2 = tc_att(xw2, gat1_att_src, gat1_att_dst)
    a2_0, a2_1, s2_0, s2_1 = sc_gat_pass(src_p, dst_p, as2, ad2, u2, xw2,
                                         z2, z1)

    xw3 = tc_post_gat_prep(a2_0[:NN], a2_1[:NN], s2_0, s2_1, gat1_b, gcn2_W,
                           dis)
    a3_0, a3_1 = sc_gcn_pass(src_p, dst_p, xw3, z2)

    xw4 = tc_post_gcn_prep(a3_0[:NN], a3_1[:NN], dis, gcn2_b, gat2_W)
    as4, ad4, u4 = tc_att(xw4, gat2_att_src, gat2_att_dst)
    a4_0, a4_1, s4_0, s4_1 = sc_gat_pass(src_p, dst_p, as4, ad4, u4, xw4,
                                         z2, z1)

    h, z = tc_final(a4_0[:NN], a4_1[:NN], s4_0, s4_1, gat2_b, out_W, out_b)
    return (h, z)


# speedup vs baseline: 1.4842x; 1.0040x over previous
import jax
import jax.numpy as jnp
from jax import lax
from jax.experimental import pallas as pl
from jax.experimental.pallas import tpu as pltpu
from jax.experimental.pallas import tpu_sc as plsc

NN = 10000
NP2 = 10240
DD = 128
CC = 64
NEG_SLOPE = 0.2

ER = 330000
NC, NS, LANES = 2, 16, 16
NW = NC * NS
KE = 128
NCH = 81
BW = NCH * KE
EPAD = BW * NW
RPS = NP2 // NS

_sc_mesh = plsc.VectorSubcoreMesh(core_axis_name="c", subcore_axis_name="s")
_sc_params = pltpu.CompilerParams(needs_layout_passes=False)


def _mm_pre_kernel(x_ref, w_ref, dis_ref, o_ref):
    xw = jnp.dot(x_ref[...], w_ref[...], preferred_element_type=jnp.float32)
    o_ref[...] = xw * dis_ref[...][:, None]


def tc_matmul_pre(x, w, dis):
    return pl.pallas_call(
        _mm_pre_kernel,
        out_shape=jax.ShapeDtypeStruct((x.shape[0], w.shape[1]), jnp.float32),
    )(x, w, dis)


def _dis_kernel(d0_ref, d1_ref, o_ref):
    d = d0_ref[...][:NN] + d1_ref[...][:NN]
    o_ref[...] = jnp.where(d > 0, lax.rsqrt(d), 0.0)


def tc_dis(deg0, deg1):
    return pl.pallas_call(
        _dis_kernel,
        out_shape=jax.ShapeDtypeStruct((NN,), jnp.float32),
    )(deg0, deg1)


def _post_gcn_prep_kernel(a0_ref, a1_ref, dis_ref, b_ref, w_ref, o_ref):
    acc = (a0_ref[...] + a1_ref[...]) * dis_ref[...][:, None]
    h = jnp.maximum(acc + b_ref[...][None, :], 0.0)
    o_ref[...] = jnp.dot(h, w_ref[...], preferred_element_type=jnp.float32)


def tc_post_gcn_prep(acc0, acc1, dis, b, w):
    return pl.pallas_call(
        _post_gcn_prep_kernel,
        out_shape=jax.ShapeDtypeStruct((NN, w.shape[1]), jnp.float32),
    )(acc0, acc1, dis, b, w)


def _att_kernel(xw_ref, asrc_ref, adst_ref, as_ref, ad_ref, u_ref):
    xw = xw_ref[...]
    a_s = jnp.sum(xw * asrc_ref[...][None, :], axis=1)
    a_d = jnp.sum(xw * adst_ref[...][None, :], axis=1)
    as_ref[...] = a_s
    ad_ref[...] = a_d
    e_ub = jnp.max(a_s) + a_d
    u_ref[...] = jnp.where(e_ub >= 0, e_ub, NEG_SLOPE * e_ub)


def tc_att(xw, a_src, a_dst):
    return pl.pallas_call(
        _att_kernel,
        out_shape=[jax.ShapeDtypeStruct((NN,), jnp.float32)] * 3,
    )(xw, a_src, a_dst)


def _post_gat_prep_kernel(a0_ref, a1_ref, s0_ref, s1_ref, b_ref, w_ref,
                          dis_ref, o_ref):
    s = (s0_ref[...][:NN] + s1_ref[...][:NN])[:, None] + 1e-16
    h = jnp.maximum((a0_ref[...] + a1_ref[...]) / s + b_ref[...][None, :], 0.0)
    xw = jnp.dot(h, w_ref[...], preferred_element_type=jnp.float32)
    o_ref[...] = xw * dis_ref[...][:, None]


def tc_post_gat_prep(acc0, acc1, s0, s1, b, w, dis):
    return pl.pallas_call(
        _post_gat_prep_kernel,
        out_shape=jax.ShapeDtypeStruct((NN, w.shape[1]), jnp.float32),
    )(acc0, acc1, s0, s1, b, w, dis)


def _final_kernel(a0_ref, a1_ref, s0_ref, s1_ref, b_ref, w_ref, ob_ref,
                  h_ref, z_ref):
    s = (s0_ref[...][:NN] + s1_ref[...][:NN])[:, None] + 1e-16
    h = jnp.maximum((a0_ref[...] + a1_ref[...]) / s + b_ref[...][None, :], 0.0)
    h_ref[...] = h
    z_ref[...] = (jnp.dot(h, w_ref[...], preferred_element_type=jnp.float32)
                  + ob_ref[...][None, :])


def tc_final(acc0, acc1, s0, s1, b, w, ob):
    return pl.pallas_call(
        _final_kernel,
        out_shape=[jax.ShapeDtypeStruct((NN, DD), jnp.float32),
                   jax.ShapeDtypeStruct((NN, CC), jnp.float32)],
    )(acc0, acc1, s0, s1, b, w, ob)


def _edge_mask(eb, j):
    ids = eb + j * LANES + lax.iota(jnp.int32, LANES)
    return ids < ER


def _deg_body(dst_hbm, z1_hbm, out0, out1, didx, ones_v, deg_sh, sem):
    c = lax.axis_index("c")
    s = lax.axis_index("s")
    wid = c * NS + s
    pltpu.sync_copy(z1_hbm, deg_sh.at[pl.ds(s * RPS, RPS)])
    for j in range(KE // LANES):
        ones_v[pl.ds(j * LANES, LANES)] = jnp.full((LANES,), 1.0, jnp.float32)
    plsc.subcore_barrier()

    def chunk(i, carry):
        eb = wid * BW + i * KE
        pltpu.sync_copy(dst_hbm.at[pl.ds(eb, KE)], didx)
        pltpu.sync_copy(ones_v, deg_sh.at[didx], add=True)
        return carry

    lax.fori_loop(0, NCH, chunk, 0)
    plsc.subcore_barrier()
    sl = pl.ds(s * RPS, RPS)

    @pl.when(c == 0)
    def _():
        pltpu.sync_copy(deg_sh.at[sl], out0.at[sl])

    @pl.when(c == 1)
    def _():
        pltpu.sync_copy(deg_sh.at[sl], out1.at[sl])


def sc_degree(dst_p, z1):
    f = pl.kernel(
        _deg_body,
        out_type=[jax.ShapeDtypeStruct((NP2,), jnp.float32)] * 2,
        mesh=_sc_mesh,
        compiler_params=_sc_params,
        scratch_types=[
            pltpu.VMEM((KE,), jnp.int32),
            pltpu.VMEM((KE,), jnp.float32),
            pltpu.VMEM_SHARED((NP2,), jnp.float32),
            pltpu.SemaphoreType.DMA,
        ],
    )
    return f(dst_p, z1)


def _gcn_body(src_hbm, dst_hbm, xw_hbm, z2_hbm, out0, out1,
              sidx, didx, rows, acc_sh, sem):
    c = lax.axis_index("c")
    s = lax.axis_index("s")
    wid = c * NS + s
    pltpu.sync_copy(z2_hbm, acc_sh.at[pl.ds(s * RPS, RPS)])
    plsc.subcore_barrier()

    def chunk(i, carry):
        eb = wid * BW + i * KE
        pltpu.sync_copy(src_hbm.at[pl.ds(eb, KE)], sidx)
        g = pltpu.async_copy(xw_hbm.at[sidx], rows, sem)
        pltpu.sync_copy(dst_hbm.at[pl.ds(eb, KE)], didx)
        g.wait()
        pltpu.sync_copy(rows, acc_sh.at[didx], add=True)
        return carry

    lax.fori_loop(0, NCH, chunk, 0)
    plsc.subcore_barrier()
    sl = pl.ds(s * RPS, RPS)

    @pl.when(c == 0)
    def _():
        pltpu.sync_copy(acc_sh.at[sl], out0.at[sl])

    @pl.when(c == 1)
    def _():
        pltpu.sync_copy(acc_sh.at[sl], out1.at[sl])


def sc_gcn_pass(src_p, dst_p, xw, z2):
    f = pl.kernel(
        _gcn_body,
        out_type=[jax.ShapeDtypeStruct((NP2, DD), jnp.float32)] * 2,
        mesh=_sc_mesh,
        compiler_params=_sc_params,
        scratch_types=[
            pltpu.VMEM((KE,), jnp.int32),
            pltpu.VMEM((KE,), jnp.int32),
            pltpu.VMEM((KE, DD), jnp.float32),
            pltpu.VMEM_SHARED((NP2, DD), jnp.float32),
            pltpu.SemaphoreType.DMA,
        ],
    )
    return f(src_p, dst_p, xw, z2)


def _gat_body(src_hbm, dst_hbm, as_hbm, ad_hbm, u_hbm, xw_hbm, z2_hbm, z1_hbm,
              out0, out1, so0, so1,
              as_v, ad_v, u_v, sidx, didx, rows, wv, acc_sh, s_sh, sem):
    c = lax.axis_index("c")
    s = lax.axis_index("s")
    wid = c * NS + s
    pltpu.sync_copy(z2_hbm, acc_sh.at[pl.ds(s * RPS, RPS)])
    pltpu.sync_copy(z1_hbm, s_sh.at[pl.ds(s * RPS, RPS)])
    pltpu.sync_copy(as_hbm, as_v)
    pltpu.sync_copy(ad_hbm, ad_v)
    pltpu.sync_copy(u_hbm, u_v)
    plsc.subcore_barrier()

    def chunk(i, carry):
        eb = wid * BW + i * KE
        pltpu.sync_copy(src_hbm.at[pl.ds(eb, KE)], sidx)
        g = pltpu.async_copy(xw_hbm.at[sidx], rows, sem)
        pltpu.sync_copy(dst_hbm.at[pl.ds(eb, KE)], didx)

        def grp(j, carry2):
            sl = pl.ds(j * LANES, LANES)
            sv = sidx[sl]
            dv = didx[sl]
            e = plsc.load_gather(as_v, [sv]) + plsc.load_gather(ad_v, [dv])
            e = jnp.where(e >= 0, e, NEG_SLOPE * e)
            ex = jnp.exp(e - plsc.load_gather(u_v, [dv]))
            ex = jnp.where(_edge_mask(eb, j), ex, 0.0)
            wv[sl] = ex
            return carry2

        lax.fori_loop(0, KE // LANES, grp, 0)
        g.wait()

        def scale(e, carry2):
            w = plsc.load_gather(wv, [jnp.full((LANES,), e, jnp.int32)])
            for jj in range(DD // LANES):
                sl = pl.ds(jj * LANES, LANES)
                rows[e, sl] = rows[e, sl] * w
            return carry2

        lax.fori_loop(0, KE, scale, 0, unroll=2)
        pltpu.sync_copy(rows, acc_sh.at[didx], add=True)
        pltpu.sync_copy(wv, s_sh.at[didx], add=True)
        return carry

    lax.fori_loop(0, NCH, chunk, 0)
    plsc.subcore_barrier()
    sl = pl.ds(s * RPS, RPS)

    @pl.when(c == 0)
    def _():
        pltpu.sync_copy(acc_sh.at[sl], out0.at[sl])
        pltpu.sync_copy(s_sh.at[sl], so0.at[sl])

    @pl.when(c == 1)
    def _():
        pltpu.sync_copy(acc_sh.at[sl], out1.at[sl])
        pltpu.sync_copy(s_sh.at[sl], so1.at[sl])


def sc_gat_pass(src_p, dst_p, a_s, a_d, u, xw, z2, z1):
    f = pl.kernel(
        _gat_body,
        out_type=[jax.ShapeDtypeStruct((NP2, DD), jnp.float32)] * 2
        + [jax.ShapeDtypeStruct((NP2,), jnp.float32)] * 2,
        mesh=_sc_mesh,
        compiler_params=_sc_params,
        scratch_types=[
            pltpu.VMEM((NN,), jnp.float32),
            pltpu.VMEM((NN,), jnp.float32),
            pltpu.VMEM((NN,), jnp.float32),
            pltpu.VMEM((KE,), jnp.int32),
            pltpu.VMEM((KE,), jnp.int32),
            pltpu.VMEM((KE, DD), jnp.float32),
            pltpu.VMEM((KE,), jnp.float32),
            pltpu.VMEM_SHARED((NP2, DD), jnp.float32),
            pltpu.VMEM_SHARED((NP2,), jnp.float32),
            pltpu.SemaphoreType.DMA,
        ],
    )
    return f(src_p, dst_p, a_s, a_d, u, xw, z2, z1)


def kernel(x, edge_index, gcn1_W, gcn1_b, gat1_W, gat1_att_src, gat1_att_dst,
           gat1_b, gcn2_W, gcn2_b, gat2_W, gat2_att_src, gat2_att_dst,
           gat2_b, out_W, out_b):
    loops = jnp.arange(NN, dtype=jnp.int32)
    spad = jnp.zeros((EPAD - ER,), jnp.int32)
    dpad = NN + (jnp.arange(EPAD - ER, dtype=jnp.int32) % (NP2 - NN))
    src_p = jnp.concatenate([edge_index[0].astype(jnp.int32), loops, spad])
    dst_p = jnp.concatenate([edge_index[1].astype(jnp.int32), loops, dpad])
    z1 = jnp.zeros((RPS,), jnp.float32)
    z2 = jnp.zeros((RPS, DD), jnp.float32)

    deg0, deg1 = sc_degree(dst_p, z1)
    dis = tc_dis(deg0, deg1)

    xw1 = tc_matmul_pre(x, gcn1_W, dis)
    a1_0, a1_1 = sc_gcn_pass(src_p, dst_p, xw1, z2)

    xw2 = tc_post_gcn_prep(a1_0[:NN], a1_1[:NN], dis, gcn1_b, gat1_W)
    as2, ad2, u2 = tc_att(xw2, gat1_att_src, gat1_att_dst)
    a2_0, a2_1, s2_0, s2_1 = sc_gat_pass(src_p, dst_p, as2, ad2, u2, xw2,
                                         z2, z1)

    xw3 = tc_post_gat_prep(a2_0[:NN], a2_1[:NN], s2_0, s2_1, gat1_b, gcn2_W,
                           dis)
    a3_0, a3_1 = sc_gcn_pass(src_p, dst_p, xw3, z2)

    xw4 = tc_post_gcn_prep(a3_0[:NN], a3_1[:NN], dis, gcn2_b, gat2_W)
    as4, ad4, u4 = tc_att(xw4, gat2_att_src, gat2_att_dst)
    a4_0, a4_1, s4_0, s4_1 = sc_gat_pass(src_p, dst_p, as4, ad4, u4, xw4,
                                         z2, z1)

    h, z = tc_final(a4_0[:NN], a4_1[:NN], s4_0, s4_1, gat2_b, out_W, out_b)
    return (h, z)


# speedup vs baseline: 1.5580x; 1.0497x over previous
import jax
import jax.numpy as jnp
from jax import lax
from jax.experimental import pallas as pl
from jax.experimental.pallas import tpu as pltpu
from jax.experimental.pallas import tpu_sc as plsc

NN = 10000
NP2 = 10240
DD = 128
CC = 64
NEG_SLOPE = 0.2

ER = 330000
NC, NS, LANES = 2, 16, 16
NW = NC * NS
KE = 128
NCH = 81
BW = NCH * KE
EPAD = BW * NW
RPS = NP2 // NS

_sc_mesh = plsc.VectorSubcoreMesh(core_axis_name="c", subcore_axis_name="s")
_sc_params = pltpu.CompilerParams(needs_layout_passes=False)


def _mm_kernel(x_ref, w_ref, o_ref):
    o_ref[...] = jnp.dot(x_ref[...], w_ref[...],
                         preferred_element_type=jnp.float32)


def tc_matmul(x, w):
    return pl.pallas_call(
        _mm_kernel,
        out_shape=jax.ShapeDtypeStruct((x.shape[0], w.shape[1]), jnp.float32),
    )(x, w)


def _dis_kernel(d0_ref, d1_ref, xw_ref, o_ref, xo_ref):
    d = d0_ref[...][:NN] + d1_ref[...][:NN]
    dis = jnp.where(d > 0, lax.rsqrt(d), 0.0)
    o_ref[...] = dis
    xo_ref[...] = xw_ref[...] * dis[:, None]


def tc_dis(deg0, deg1, xw1r):
    return pl.pallas_call(
        _dis_kernel,
        out_shape=[jax.ShapeDtypeStruct((NN,), jnp.float32),
                   jax.ShapeDtypeStruct((NN, DD), jnp.float32)],
    )(deg0, deg1, xw1r)


def _post_gcn_prep_kernel(a0_ref, a1_ref, dis_ref, b_ref, w_ref,
                          asrc_ref, adst_ref, o_ref, as_ref, ad_ref, u_ref):
    acc = (a0_ref[...] + a1_ref[...]) * dis_ref[...][:, None]
    h = jnp.maximum(acc + b_ref[...][None, :], 0.0)
    xw = jnp.dot(h, w_ref[...], preferred_element_type=jnp.float32)
    o_ref[...] = xw
    a_s = jnp.sum(xw * asrc_ref[...][None, :], axis=1)
    a_d = jnp.sum(xw * adst_ref[...][None, :], axis=1)
    as_ref[...] = a_s
    ad_ref[...] = a_d
    e_ub = jnp.max(a_s) + a_d
    u_ref[...] = jnp.where(e_ub >= 0, e_ub, NEG_SLOPE * e_ub)


def tc_post_gcn_prep(acc0, acc1, dis, b, w, a_src, a_dst):
    return pl.pallas_call(
        _post_gcn_prep_kernel,
        out_shape=[jax.ShapeDtypeStruct((NN, w.shape[1]), jnp.float32)]
        + [jax.ShapeDtypeStruct((NN,), jnp.float32)] * 3,
    )(acc0, acc1, dis, b, w, a_src, a_dst)


def _att_kernel(xw_ref, asrc_ref, adst_ref, as_ref, ad_ref, u_ref):
    xw = xw_ref[...]
    a_s = jnp.sum(xw * asrc_ref[...][None, :], axis=1)
    a_d = jnp.sum(xw * adst_ref[...][None, :], axis=1)
    as_ref[...] = a_s
    ad_ref[...] = a_d
    e_ub = jnp.max(a_s) + a_d
    u_ref[...] = jnp.where(e_ub >= 0, e_ub, NEG_SLOPE * e_ub)


def tc_att(xw, a_src, a_dst):
    return pl.pallas_call(
        _att_kernel,
        out_shape=[jax.ShapeDtypeStruct((NN,), jnp.float32)] * 3,
    )(xw, a_src, a_dst)


def _post_gat_prep_kernel(a0_ref, a1_ref, s0_ref, s1_ref, b_ref, w_ref,
                          dis_ref, o_ref):
    s = (s0_ref[...][:NN] + s1_ref[...][:NN])[:, None] + 1e-16
    h = jnp.maximum((a0_ref[...] + a1_ref[...]) / s + b_ref[...][None, :], 0.0)
    xw = jnp.dot(h, w_ref[...], preferred_element_type=jnp.float32)
    o_ref[...] = xw * dis_ref[...][:, None]


def tc_post_gat_prep(acc0, acc1, s0, s1, b, w, dis):
    return pl.pallas_call(
        _post_gat_prep_kernel,
        out_shape=jax.ShapeDtypeStruct((NN, w.shape[1]), jnp.float32),
    )(acc0, acc1, s0, s1, b, w, dis)


def _final_kernel(a0_ref, a1_ref, s0_ref, s1_ref, b_ref, w_ref, ob_ref,
                  h_ref, z_ref):
    s = (s0_ref[...][:NN] + s1_ref[...][:NN])[:, None] + 1e-16
    h = jnp.maximum((a0_ref[...] + a1_ref[...]) / s + b_ref[...][None, :], 0.0)
    h_ref[...] = h
    z_ref[...] = (jnp.dot(h, w_ref[...], preferred_element_type=jnp.float32)
                  + ob_ref[...][None, :])


def tc_final(acc0, acc1, s0, s1, b, w, ob):
    return pl.pallas_call(
        _final_kernel,
        out_shape=[jax.ShapeDtypeStruct((NN, DD), jnp.float32),
                   jax.ShapeDtypeStruct((NN, CC), jnp.float32)],
    )(acc0, acc1, s0, s1, b, w, ob)


def _edge_mask(eb, j):
    ids = eb + j * LANES + lax.iota(jnp.int32, LANES)
    return ids < ER


def _deg_body(dst_hbm, z1_hbm, out0, out1, didx, ones_v, deg_sh, sem):
    c = lax.axis_index("c")
    s = lax.axis_index("s")
    wid = c * NS + s
    pltpu.sync_copy(z1_hbm, deg_sh.at[pl.ds(s * RPS, RPS)])
    for j in range(KE // LANES):
        ones_v[pl.ds(j * LANES, LANES)] = jnp.full((LANES,), 1.0, jnp.float32)
    plsc.subcore_barrier()

    def chunk(i, carry):
        eb = wid * BW + i * KE
        pltpu.sync_copy(dst_hbm.at[pl.ds(eb, KE)], didx)
        pltpu.sync_copy(ones_v, deg_sh.at[didx], add=True)
        return carry

    lax.fori_loop(0, NCH, chunk, 0)
    plsc.subcore_barrier()
    sl = pl.ds(s * RPS, RPS)

    @pl.when(c == 0)
    def _():
        pltpu.sync_copy(deg_sh.at[sl], out0.at[sl])

    @pl.when(c == 1)
    def _():
        pltpu.sync_copy(deg_sh.at[sl], out1.at[sl])


def sc_degree(dst_p, z1):
    f = pl.kernel(
        _deg_body,
        out_type=[jax.ShapeDtypeStruct((NP2,), jnp.float32)] * 2,
        mesh=_sc_mesh,
        compiler_params=_sc_params,
        scratch_types=[
            pltpu.VMEM((KE,), jnp.int32),
            pltpu.VMEM((KE,), jnp.float32),
            pltpu.VMEM_SHARED((NP2,), jnp.float32),
            pltpu.SemaphoreType.DMA,
        ],
    )
    return f(dst_p, z1)


def _gcn_body(src_hbm, dst_hbm, xw_hbm, z2_hbm, out0, out1,
              sidx, didx, rows, acc_sh, sem):
    c = lax.axis_index("c")
    s = lax.axis_index("s")
    wid = c * NS + s
    pltpu.sync_copy(z2_hbm, acc_sh.at[pl.ds(s * RPS, RPS)])
    plsc.subcore_barrier()

    def chunk(i, carry):
        eb = wid * BW + i * KE
        pltpu.sync_copy(src_hbm.at[pl.ds(eb, KE)], sidx)
        g = pltpu.async_copy(xw_hbm.at[sidx], rows, sem)
        pltpu.sync_copy(dst_hbm.at[pl.ds(eb, KE)], didx)
        g.wait()
        pltpu.sync_copy(rows, acc_sh.at[didx], add=True)
        return carry

    lax.fori_loop(0, NCH, chunk, 0)
    plsc.subcore_barrier()
    sl = pl.ds(s * RPS, RPS)

    @pl.when(c == 0)
    def _():
        pltpu.sync_copy(acc_sh.at[sl], out0.at[sl])

    @pl.when(c == 1)
    def _():
        pltpu.sync_copy(acc_sh.at[sl], out1.at[sl])


def sc_gcn_pass(src_p, dst_p, xw, z2):
    f = pl.kernel(
        _gcn_body,
        out_type=[jax.ShapeDtypeStruct((NP2, DD), jnp.float32)] * 2,
        mesh=_sc_mesh,
        compiler_params=_sc_params,
        scratch_types=[
            pltpu.VMEM((KE,), jnp.int32),
            pltpu.VMEM((KE,), jnp.int32),
            pltpu.VMEM((KE, DD), jnp.float32),
            pltpu.VMEM_SHARED((NP2, DD), jnp.float32),
            pltpu.SemaphoreType.DMA,
        ],
    )
    return f(src_p, dst_p, xw, z2)


def _gat_body(src_hbm, dst_hbm, as_hbm, ad_hbm, u_hbm, xw_hbm, z2_hbm, z1_hbm,
              out0, out1, so0, so1,
              as_v, ad_v, u_v, sidx, didx, rows, wv, acc_sh, s_sh, sem):
    c = lax.axis_index("c")
    s = lax.axis_index("s")
    wid = c * NS + s
    pltpu.sync_copy(z2_hbm, acc_sh.at[pl.ds(s * RPS, RPS)])
    pltpu.sync_copy(z1_hbm, s_sh.at[pl.ds(s * RPS, RPS)])
    pltpu.sync_copy(as_hbm, as_v)
    pltpu.sync_copy(ad_hbm, ad_v)
    pltpu.sync_copy(u_hbm, u_v)
    plsc.subcore_barrier()

    def chunk(i, carry):
        eb = wid * BW + i * KE
        pltpu.sync_copy(src_hbm.at[pl.ds(eb, KE)], sidx)
        g = pltpu.async_copy(xw_hbm.at[sidx], rows, sem)
        pltpu.sync_copy(dst_hbm.at[pl.ds(eb, KE)], didx)

        def grp(j, carry2):
            sl = pl.ds(j * LANES, LANES)
            sv = sidx[sl]
            dv = didx[sl]
            e = plsc.load_gather(as_v, [sv]) + plsc.load_gather(ad_v, [dv])
            e = jnp.where(e >= 0, e, NEG_SLOPE * e)
            ex = jnp.exp(e - plsc.load_gather(u_v, [dv]))
            ex = jnp.where(_edge_mask(eb, j), ex, 0.0)
            wv[sl] = ex
            return carry2

        lax.fori_loop(0, KE // LANES, grp, 0)
        g.wait()

        def scale(e, carry2):
            w = plsc.load_gather(wv, [jnp.full((LANES,), e, jnp.int32)])
            for jj in range(DD // LANES):
                sl = pl.ds(jj * LANES, LANES)
                rows[e, sl] = rows[e, sl] * w
            return carry2

        lax.fori_loop(0, KE, scale, 0, unroll=2)
        pltpu.sync_copy(rows, acc_sh.at[didx], add=True)
        pltpu.sync_copy(wv, s_sh.at[didx], add=True)
        return carry

    lax.fori_loop(0, NCH, chunk, 0)
    plsc.subcore_barrier()
    sl = pl.ds(s * RPS, RPS)

    @pl.when(c == 0)
    def _():
        pltpu.sync_copy(acc_sh.at[sl], out0.at[sl])
        pltpu.sync_copy(s_sh.at[sl], so0.at[sl])

    @pl.when(c == 1)
    def _():
        pltpu.sync_copy(acc_sh.at[sl], out1.at[sl])
        pltpu.sync_copy(s_sh.at[sl], so1.at[sl])


def sc_gat_pass(src_p, dst_p, a_s, a_d, u, xw, z2, z1):
    f = pl.kernel(
        _gat_body,
        out_type=[jax.ShapeDtypeStruct((NP2, DD), jnp.float32)] * 2
        + [jax.ShapeDtypeStruct((NP2,), jnp.float32)] * 2,
        mesh=_sc_mesh,
        compiler_params=_sc_params,
        scratch_types=[
            pltpu.VMEM((NN,), jnp.float32),
            pltpu.VMEM((NN,), jnp.float32),
            pltpu.VMEM((NN,), jnp.float32),
            pltpu.VMEM((KE,), jnp.int32),
            pltpu.VMEM((KE,), jnp.int32),
            pltpu.VMEM((KE, DD), jnp.float32),
            pltpu.VMEM((KE,), jnp.float32),
            pltpu.VMEM_SHARED((NP2, DD), jnp.float32),
            pltpu.VMEM_SHARED((NP2,), jnp.float32),
            pltpu.SemaphoreType.DMA,
        ],
    )
    return f(src_p, dst_p, a_s, a_d, u, xw, z2, z1)


def kernel(x, edge_index, gcn1_W, gcn1_b, gat1_W, gat1_att_src, gat1_att_dst,
           gat1_b, gcn2_W, gcn2_b, gat2_W, gat2_att_src, gat2_att_dst,
           gat2_b, out_W, out_b):
    loops = jnp.arange(NN, dtype=jnp.int32)
    spad = jnp.zeros((EPAD - ER,), jnp.int32)
    dpad = NN + (jnp.arange(EPAD - ER, dtype=jnp.int32) % (NP2 - NN))
    src_p = jnp.concatenate([edge_index[0].astype(jnp.int32), loops, spad])
    dst_p = jnp.concatenate([edge_index[1].astype(jnp.int32), loops, dpad])
    z1 = jnp.zeros((RPS,), jnp.float32)
    z2 = jnp.zeros((RPS, DD), jnp.float32)

    xw1r = tc_matmul(x, gcn1_W)
    deg0, deg1 = sc_degree(dst_p, z1)
    dis, xw1 = tc_dis(deg0, deg1, xw1r)

    a1_0, a1_1 = sc_gcn_pass(src_p, dst_p, xw1, z2)

    xw2, as2, ad2, u2 = tc_post_gcn_prep(a1_0[:NN], a1_1[:NN], dis, gcn1_b,
                                         gat1_W, gat1_att_src, gat1_att_dst)
    a2_0, a2_1, s2_0, s2_1 = sc_gat_pass(src_p, dst_p, as2, ad2, u2, xw2,
                                         z2, z1)

    xw3 = tc_post_gat_prep(a2_0[:NN], a2_1[:NN], s2_0, s2_1, gat1_b, gcn2_W,
                           dis)
    a3_0, a3_1 = sc_gcn_pass(src_p, dst_p, xw3, z2)

    xw4, as4, ad4, u4 = tc_post_gcn_prep(a3_0[:NN], a3_1[:NN], dis, gcn2_b,
                                         gat2_W, gat2_att_src, gat2_att_dst)
    a4_0, a4_1, s4_0, s4_1 = sc_gat_pass(src_p, dst_p, as4, ad4, u4, xw4,
                                         z2, z1)

    h, z = tc_final(a4_0[:NN], a4_1[:NN], s4_0, s4_1, gat2_b, out_W, out_b)
    return (h, z)


# speedup vs baseline: 1.5593x; 1.0008x over previous
import jax
import jax.numpy as jnp
from jax import lax
from jax.experimental import pallas as pl
from jax.experimental.pallas import tpu as pltpu
from jax.experimental.pallas import tpu_sc as plsc

NN = 10000
NP2 = 10240
DD = 128
CC = 64
NEG_SLOPE = 0.2

ER = 330000
NC, NS, LANES = 2, 16, 16
NW = NC * NS
KE = 128
NCH = 81
BW = NCH * KE
EPAD = BW * NW
RPS = NP2 // NS

_sc_mesh = plsc.VectorSubcoreMesh(core_axis_name="c", subcore_axis_name="s")
_sc_params = pltpu.CompilerParams(needs_layout_passes=False)


def _mm_kernel(x_ref, w_ref, o_ref):
    o_ref[...] = jnp.dot(x_ref[...], w_ref[...],
                         preferred_element_type=jnp.float32)


def tc_matmul(x, w):
    return pl.pallas_call(
        _mm_kernel,
        out_shape=jax.ShapeDtypeStruct((x.shape[0], w.shape[1]), jnp.float32),
    )(x, w)


def _dis_kernel(d0_ref, d1_ref, xw_ref, o_ref, xo_ref):
    d = d0_ref[...][:NN] + d1_ref[...][:NN]
    dis = jnp.where(d > 0, lax.rsqrt(d), 0.0)
    o_ref[...] = dis
    xo_ref[...] = xw_ref[...] * dis[:, None]


def tc_dis(deg0, deg1, xw1r):
    return pl.pallas_call(
        _dis_kernel,
        out_shape=[jax.ShapeDtypeStruct((NN,), jnp.float32),
                   jax.ShapeDtypeStruct((NN, DD), jnp.float32)],
    )(deg0, deg1, xw1r)


def _post_gcn_prep_kernel(a0_ref, a1_ref, dis_ref, b_ref, w_ref,
                          asrc_ref, adst_ref, o_ref, as_ref, ad_ref, u_ref):
    acc = (a0_ref[...] + a1_ref[...]) * dis_ref[...][:, None]
    h = jnp.maximum(acc + b_ref[...][None, :], 0.0)
    xw = jnp.dot(h, w_ref[...], preferred_element_type=jnp.float32)
    o_ref[...] = xw
    a_s = jnp.sum(xw * asrc_ref[...][None, :], axis=1)
    a_d = jnp.sum(xw * adst_ref[...][None, :], axis=1)
    as_ref[...] = a_s
    ad_ref[...] = a_d
    e_ub = jnp.max(a_s) + a_d
    u_ref[...] = jnp.where(e_ub >= 0, e_ub, NEG_SLOPE * e_ub)


def tc_post_gcn_prep(acc0, acc1, dis, b, w, a_src, a_dst):
    return pl.pallas_call(
        _post_gcn_prep_kernel,
        out_shape=[jax.ShapeDtypeStruct((NN, w.shape[1]), jnp.float32)]
        + [jax.ShapeDtypeStruct((NN,), jnp.float32)] * 3,
    )(acc0, acc1, dis, b, w, a_src, a_dst)


def _post_gat_prep_kernel(a0_ref, a1_ref, s0_ref, s1_ref, b_ref, w_ref,
                          dis_ref, o_ref):
    s = (s0_ref[...][:NN] + s1_ref[...][:NN])[:, None] + 1e-16
    h = jnp.maximum((a0_ref[...] + a1_ref[...]) / s + b_ref[...][None, :], 0.0)
    xw = jnp.dot(h, w_ref[...], preferred_element_type=jnp.float32)
    o_ref[...] = xw * dis_ref[...][:, None]


def tc_post_gat_prep(acc0, acc1, s0, s1, b, w, dis):
    return pl.pallas_call(
        _post_gat_prep_kernel,
        out_shape=jax.ShapeDtypeStruct((NN, w.shape[1]), jnp.float32),
    )(acc0, acc1, s0, s1, b, w, dis)


def _final_kernel(a0_ref, a1_ref, s0_ref, s1_ref, b_ref, w_ref, ob_ref,
                  h_ref, z_ref):
    s = (s0_ref[...][:NN] + s1_ref[...][:NN])[:, None] + 1e-16
    h = jnp.maximum((a0_ref[...] + a1_ref[...]) / s + b_ref[...][None, :], 0.0)
    h_ref[...] = h
    z_ref[...] = (jnp.dot(h, w_ref[...], preferred_element_type=jnp.float32)
                  + ob_ref[...][None, :])


def tc_final(acc0, acc1, s0, s1, b, w, ob):
    return pl.pallas_call(
        _final_kernel,
        out_shape=[jax.ShapeDtypeStruct((NN, DD), jnp.float32),
                   jax.ShapeDtypeStruct((NN, CC), jnp.float32)],
    )(acc0, acc1, s0, s1, b, w, ob)


def _edge_mask(eb, j):
    ids = eb + j * LANES + lax.iota(jnp.int32, LANES)
    return ids < ER


def _deg_body(dst_hbm, z1_hbm, out0, out1, didx, ones_v, deg_sh, sem):
    c = lax.axis_index("c")
    s = lax.axis_index("s")
    wid = c * NS + s
    pltpu.sync_copy(z1_hbm, deg_sh.at[pl.ds(s * RPS, RPS)])
    for j in range(KE // LANES):
        ones_v[pl.ds(j * LANES, LANES)] = jnp.full((LANES,), 1.0, jnp.float32)
    plsc.subcore_barrier()

    def chunk(i, carry):
        eb = wid * BW + i * KE
        pltpu.sync_copy(dst_hbm.at[pl.ds(eb, KE)], didx)
        pltpu.sync_copy(ones_v, deg_sh.at[didx], add=True)
        return carry

    lax.fori_loop(0, NCH, chunk, 0)
    plsc.subcore_barrier()
    sl = pl.ds(s * RPS, RPS)

    @pl.when(c == 0)
    def _():
        pltpu.sync_copy(deg_sh.at[sl], out0.at[sl])

    @pl.when(c == 1)
    def _():
        pltpu.sync_copy(deg_sh.at[sl], out1.at[sl])


def sc_degree(dst_p, z1):
    f = pl.kernel(
        _deg_body,
        out_type=[jax.ShapeDtypeStruct((NP2,), jnp.float32)] * 2,
        mesh=_sc_mesh,
        compiler_params=_sc_params,
        scratch_types=[
            pltpu.VMEM((KE,), jnp.int32),
            pltpu.VMEM((KE,), jnp.float32),
            pltpu.VMEM_SHARED((NP2,), jnp.float32),
            pltpu.SemaphoreType.DMA,
        ],
    )
    return f(dst_p, z1)


def _gcn_body(src_hbm, dst_hbm, xw_hbm, z2_hbm, out0, out1,
              sidx, didx, rows, acc_sh, sem):
    c = lax.axis_index("c")
    s = lax.axis_index("s")
    wid = c * NS + s
    pltpu.sync_copy(z2_hbm, acc_sh.at[pl.ds(s * RPS, RPS)])
    plsc.subcore_barrier()

    def chunk(i, carry):
        eb = wid * BW + i * KE
        pltpu.sync_copy(src_hbm.at[pl.ds(eb, KE)], sidx)
        g = pltpu.async_copy(xw_hbm.at[sidx], rows, sem)
        pltpu.sync_copy(dst_hbm.at[pl.ds(eb, KE)], didx)
        g.wait()
        pltpu.sync_copy(rows, acc_sh.at[didx], add=True)
        return carry

    lax.fori_loop(0, NCH, chunk, 0)
    plsc.subcore_barrier()
    sl = pl.ds(s * RPS, RPS)

    @pl.when(c == 0)
    def _():
        pltpu.sync_copy(acc_sh.at[sl], out0.at[sl])

    @pl.when(c == 1)
    def _():
        pltpu.sync_copy(acc_sh.at[sl], out1.at[sl])


def sc_gcn_pass(src_p, dst_p, xw, z2):
    f = pl.kernel(
        _gcn_body,
        out_type=[jax.ShapeDtypeStruct((NP2, DD), jnp.float32)] * 2,
        mesh=_sc_mesh,
        compiler_params=_sc_params,
        scratch_types=[
            pltpu.VMEM((KE,), jnp.int32),
            pltpu.VMEM((KE,), jnp.int32),
            pltpu.VMEM((KE, DD), jnp.float32),
            pltpu.VMEM_SHARED((NP2, DD), jnp.float32),
            pltpu.SemaphoreType.DMA,
        ],
    )
    return f(src_p, dst_p, xw, z2)


def _gat_body(src_hbm, dst_hbm, as_hbm, ad_hbm, u_hbm, xw_hbm, z2_hbm, z1_hbm,
              out0, out1, so0, so1,
              as_v, ad_v, u_v, sidx, didx, rows, wv, acc_sh, s_sh, sem):
    c = lax.axis_index("c")
    s = lax.axis_index("s")
    wid = c * NS + s
    pltpu.sync_copy(z2_hbm, acc_sh.at[pl.ds(s * RPS, RPS)])
    pltpu.sync_copy(z1_hbm, s_sh.at[pl.ds(s * RPS, RPS)])
    pltpu.sync_copy(as_hbm, as_v)
    pltpu.sync_copy(ad_hbm, ad_v)
    pltpu.sync_copy(u_hbm, u_v)
    plsc.subcore_barrier()

    def chunk(i, carry):
        eb = wid * BW + i * KE
        pltpu.sync_copy(src_hbm.at[pl.ds(eb, KE)], sidx)
        g = pltpu.async_copy(xw_hbm.at[sidx], rows, sem)
        pltpu.sync_copy(dst_hbm.at[pl.ds(eb, KE)], didx)

        def grp(j, carry2):
            sl = pl.ds(j * LANES, LANES)
            sv = sidx[sl]
            dv = didx[sl]
            e = plsc.load_gather(as_v, [sv]) + plsc.load_gather(ad_v, [dv])
            e = jnp.where(e >= 0, e, NEG_SLOPE * e)
            ex = jnp.exp(e - plsc.load_gather(u_v, [dv]))
            ex = jnp.where(_edge_mask(eb, j), ex, 0.0)
            wv[sl] = ex
            return carry2

        lax.fori_loop(0, KE // LANES, grp, 0)
        g.wait()

        def scale(e, carry2):
            w = plsc.load_gather(wv, [jnp.full((LANES,), e, jnp.int32)])
            for jj in range(DD // LANES):
                sl = pl.ds(jj * LANES, LANES)
                rows[e, sl] = rows[e, sl] * w
            return carry2

        lax.fori_loop(0, KE, scale, 0, unroll=2)
        pltpu.sync_copy(rows, acc_sh.at[didx], add=True)
        pltpu.sync_copy(wv, s_sh.at[didx], add=True)
        return carry

    lax.fori_loop(0, NCH, chunk, 0)
    plsc.subcore_barrier()
    sl = pl.ds(s * RPS, RPS)

    @pl.when(c == 0)
    def _():
        pltpu.sync_copy(acc_sh.at[sl], out0.at[sl])
        pltpu.sync_copy(s_sh.at[sl], so0.at[sl])

    @pl.when(c == 1)
    def _():
        pltpu.sync_copy(acc_sh.at[sl], out1.at[sl])
        pltpu.sync_copy(s_sh.at[sl], so1.at[sl])


def sc_gat_pass(src_p, dst_p, a_s, a_d, u, xw, z2, z1):
    f = pl.kernel(
        _gat_body,
        out_type=[jax.ShapeDtypeStruct((NP2, DD), jnp.float32)] * 2
        + [jax.ShapeDtypeStruct((NP2,), jnp.float32)] * 2,
        mesh=_sc_mesh,
        compiler_params=_sc_params,
        scratch_types=[
            pltpu.VMEM((NN,), jnp.float32),
            pltpu.VMEM((NN,), jnp.float32),
            pltpu.VMEM((NN,), jnp.float32),
            pltpu.VMEM((KE,), jnp.int32),
            pltpu.VMEM((KE,), jnp.int32),
            pltpu.VMEM((KE, DD), jnp.float32),
            pltpu.VMEM((KE,), jnp.float32),
            pltpu.VMEM_SHARED((NP2, DD), jnp.float32),
            pltpu.VMEM_SHARED((NP2,), jnp.float32),
            pltpu.SemaphoreType.DMA,
        ],
    )
    return f(src_p, dst_p, a_s, a_d, u, xw, z2, z1)


def kernel(x, edge_index, gcn1_W, gcn1_b, gat1_W, gat1_att_src, gat1_att_dst,
           gat1_b, gcn2_W, gcn2_b, gat2_W, gat2_att_src, gat2_att_dst,
           gat2_b, out_W, out_b):
    loops = jnp.arange(NN, dtype=jnp.int32)
    spad = jnp.zeros((EPAD - ER,), jnp.int32)
    dpad = NN + (jnp.arange(EPAD - ER, dtype=jnp.int32) % (NP2 - NN))
    src_p = jnp.concatenate([edge_index[0].astype(jnp.int32), loops, spad])
    dst_p = jnp.concatenate([edge_index[1].astype(jnp.int32), loops, dpad])
    z1 = jnp.zeros((RPS,), jnp.float32)
    z2 = jnp.zeros((RPS, DD), jnp.float32)

    xw1r = tc_matmul(x, gcn1_W)
    deg0, deg1 = sc_degree(dst_p, z1)
    dis, xw1 = tc_dis(deg0, deg1, xw1r)

    a1_0, a1_1 = sc_gcn_pass(src_p, dst_p, xw1, z2)

    xw2, as2, ad2, u2 = tc_post_gcn_prep(a1_0[:NN], a1_1[:NN], dis, gcn1_b,
                                         gat1_W, gat1_att_src, gat1_att_dst)
    a2_0, a2_1, s2_0, s2_1 = sc_gat_pass(src_p, dst_p, as2, ad2, u2, xw2,
                                         z2, z1)

    xw3 = tc_post_gat_prep(a2_0[:NN], a2_1[:NN], s2_0, s2_1, gat1_b, gcn2_W,
                           dis)
    a3_0, a3_1 = sc_gcn_pass(src_p, dst_p, xw3, z2)

    xw4, as4, ad4, u4 = tc_post_gcn_prep(a3_0[:NN], a3_1[:NN], dis, gcn2_b,
                                         gat2_W, gat2_att_src, gat2_att_dst)
    a4_0, a4_1, s4_0, s4_1 = sc_gat_pass(src_p, dst_p, as4, ad4, u4, xw4,
                                         z2, z1)

    h, z = tc_final(a4_0[:NN], a4_1[:NN], s4_0, s4_1, gat2_b, out_W, out_b)
    return (h, z)
